# Initial kernel scaffold; baseline (speedup 1.0000x reference)
import jax
import jax.numpy as jnp
from jax.experimental import pallas as pl


def kernel(img_feats, verts_padded, edges_packed, params):
    raise NotImplementedError("write your pallas kernel here")


# speedup vs baseline: 2.0043x; 2.0043x over previous
import functools

import jax
import jax.numpy as jnp
from jax import lax
from jax.experimental import pallas as pl
from jax.experimental.pallas import tpu as pltpu
from jax.experimental.pallas import tpu_sc as plsc

N, V, E = 10, 10000, 300000
C_IMG, H_IMG, W_IMG = 256, 32, 32
HIDDEN = 128
NUM_STAGES = 3
STAGE_DEPTH = 3

VTOT = N * V
VP = 100352
RB = 512
NROW = VP // RB
E2 = 2 * E
SC_TILES = 16
EB = 128
EBATCH = 512
E_PER_TILE = 38912
E2P = SC_TILES * E_PER_TILE
NBATCH = E_PER_TILE // EBATCH
ACC_ROWS = VP
DUMMY_ROW = VTOT
VPT = VP // SC_TILES
ZCHUNK = 112
NCHUNKS = 8
TAPS_TOT = 4 * VP
TAP_ROWS_PER_TILE = TAPS_TOT // 32
TAP_BATCHES = TAP_ROWS_PER_TILE // EB


def _mm_imgproj_body(x_ref, w_ref, o_ref):
    o_ref[...] = jnp.dot(x_ref[...], w_ref[...],
                         preferred_element_type=jnp.float32)


def _tc_imgproj(img_t, w):
    return pl.pallas_call(
        _mm_imgproj_body,
        grid=(10240 // RB,),
        in_specs=[pl.BlockSpec((RB, C_IMG), lambda i: (i, 0)),
                  pl.BlockSpec((C_IMG, 384), lambda i: (0, 0))],
        out_specs=pl.BlockSpec((RB, 384), lambda i: (i, 0)),
        out_shape=jax.ShapeDtypeStruct((10240, 384), jnp.float32),
    )(img_t, w)


def _prep_body(xs_ref, ys_ref, idx_ref, w_ref):
    gx = xs_ref[...]
    gy = ys_ref[...]
    x = (gx + 1.0) * ((W_IMG - 1) / 2.0)
    y = (1.0 - gy) * ((H_IMG - 1) / 2.0)
    x0 = jnp.floor(x)
    y0 = jnp.floor(y)
    wx1 = x - x0
    wx0 = 1.0 - wx1
    wy1 = y - y0
    wy0 = 1.0 - wy1
    vid = lax.broadcasted_iota(jnp.int32, (784, 128), 0) * 128 + \
        lax.broadcasted_iota(jnp.int32, (784, 128), 1)
    n = jnp.clip(vid // V, 0, N - 1)
    for t, (ix, iy, wt) in enumerate((
            (x0, y0, wx0 * wy0), (x0 + 1.0, y0, wx1 * wy0),
            (x0, y0 + 1.0, wx0 * wy1), (x0 + 1.0, y0 + 1.0, wx1 * wy1))):
        valid = ((ix >= 0.0) & (ix <= W_IMG - 1.0)
                 & (iy >= 0.0) & (iy <= H_IMG - 1.0))
        ixc = jnp.clip(ix, 0.0, W_IMG - 1.0).astype(jnp.int32)
        iyc = jnp.clip(iy, 0.0, H_IMG - 1.0).astype(jnp.int32)
        idx_ref[t] = n * (H_IMG * W_IMG) + iyc * W_IMG + ixc
        w_ref[t] = jnp.where(valid, wt, 0.0)


def _tc_prep(xs, ys):
    return pl.pallas_call(
        _prep_body,
        in_specs=[pl.BlockSpec((784, 128), lambda: (0, 0)),
                  pl.BlockSpec((784, 128), lambda: (0, 0))],
        out_specs=[pl.BlockSpec((4, 784, 128), lambda: (0, 0, 0)),
                   pl.BlockSpec((4, 784, 128), lambda: (0, 0, 0))],
        out_shape=[jax.ShapeDtypeStruct((4, 784, 128), jnp.int32),
                   jax.ShapeDtypeStruct((4, 784, 128), jnp.float32)],
    )(xs, ys)


def _va_body(t0, t1, t2, t3, w_ref, b_ref, o_ref):
    w = w_ref[...]
    acc = t0[...] * w[:, 0:1] + t1[...] * w[:, 1:2]
    acc += t2[...] * w[:, 2:3] + t3[...] * w[:, 3:4]
    o_ref[...] = jnp.maximum(acc + b_ref[...], 0.0)


def _tc_va(taps_flat, w8, bb):
    specs = [pl.BlockSpec((RB, 128), functools.partial(
        lambda i, t: (t * NROW + i, 0), t=t)) for t in range(4)]
    return pl.pallas_call(
        _va_body,
        grid=(NROW,),
        in_specs=specs + [pl.BlockSpec((RB, 8), lambda i: (i, 0)),
                          pl.BlockSpec((1, 128), lambda i: (0, 0))],
        out_specs=pl.BlockSpec((RB, 128), lambda i: (i, 0)),
        out_shape=jax.ShapeDtypeStruct((VP, 128), jnp.float32),
    )(taps_flat, taps_flat, taps_flat, taps_flat, w8, bb)


def _gconv_first_body(a_ref, p_ref, f_ref, w_ref, b_ref, o0_ref, o1_ref):
    xin = jnp.concatenate([a_ref[...], p_ref[...], f_ref[...]], axis=1)
    out = jnp.dot(xin, w_ref[...], preferred_element_type=jnp.float32)
    out += b_ref[...]
    o0_ref[...] = out[:, :HIDDEN]
    o1_ref[...] = out[:, HIDDEN:]


def _tc_gconv_first(va, pos8, vfeat, w, b):
    return pl.pallas_call(
        _gconv_first_body,
        grid=(NROW,),
        in_specs=[pl.BlockSpec((RB, 128), lambda i: (i, 0)),
                  pl.BlockSpec((RB, 8), lambda i: (i, 0)),
                  pl.BlockSpec((RB, 128), lambda i: (i, 0)),
                  pl.BlockSpec((264, 256), lambda i: (0, 0)),
                  pl.BlockSpec((1, 256), lambda i: (0, 0))],
        out_specs=[pl.BlockSpec((RB, 128), lambda i: (i, 0)),
                   pl.BlockSpec((RB, 128), lambda i: (i, 0))],
        out_shape=[jax.ShapeDtypeStruct((VP, 128), jnp.float32),
                   jax.ShapeDtypeStruct((VP, 128), jnp.float32)],
    )(va, pos8, vfeat, w, b)


def _gconv_first_s0_body(a_ref, p_ref, w_ref, b_ref, o0_ref, o1_ref):
    xin = jnp.concatenate([a_ref[...], p_ref[...]], axis=1)
    out = jnp.dot(xin, w_ref[...], preferred_element_type=jnp.float32)
    out += b_ref[...]
    o0_ref[...] = out[:, :HIDDEN]
    o1_ref[...] = out[:, HIDDEN:]


def _tc_gconv_first_s0(va, pos8, w, b):
    return pl.pallas_call(
        _gconv_first_s0_body,
        grid=(NROW,),
        in_specs=[pl.BlockSpec((RB, 128), lambda i: (i, 0)),
                  pl.BlockSpec((RB, 8), lambda i: (i, 0)),
                  pl.BlockSpec((136, 256), lambda i: (0, 0)),
                  pl.BlockSpec((1, 256), lambda i: (0, 0))],
        out_specs=[pl.BlockSpec((RB, 128), lambda i: (i, 0)),
                   pl.BlockSpec((RB, 128), lambda i: (i, 0))],
        out_shape=[jax.ShapeDtypeStruct((VP, 128), jnp.float32),
                   jax.ShapeDtypeStruct((VP, 128), jnp.float32)],
    )(va, pos8, w, b)


def _gconv_mid_body(x0_ref, g_ref, p_ref, w_ref, b_ref, o0_ref, o1_ref):
    nopos = jnp.maximum(x0_ref[...] + g_ref[...], 0.0)
    xin = jnp.concatenate([nopos, p_ref[...]], axis=1)
    out = jnp.dot(xin, w_ref[...], preferred_element_type=jnp.float32)
    out += b_ref[...]
    o0_ref[...] = out[:, :HIDDEN]
    o1_ref[...] = out[:, HIDDEN:]


def _tc_gconv_mid(xw0, agg, pos8, w, b):
    return pl.pallas_call(
        _gconv_mid_body,
        grid=(NROW,),
        in_specs=[pl.BlockSpec((RB, 128), lambda i: (i, 0)),
                  pl.BlockSpec((RB, 128), lambda i: (i, 0)),
                  pl.BlockSpec((RB, 8), lambda i: (i, 0)),
                  pl.BlockSpec((136, 256), lambda i: (0, 0)),
                  pl.BlockSpec((1, 256), lambda i: (0, 0))],
        out_specs=[pl.BlockSpec((RB, 128), lambda i: (i, 0)),
                   pl.BlockSpec((RB, 128), lambda i: (i, 0))],
        out_shape=[jax.ShapeDtypeStruct((VP, 128), jnp.float32),
                   jax.ShapeDtypeStruct((VP, 128), jnp.float32)],
    )(xw0, agg, pos8, w, b)


def _offset_body(x0_ref, g_ref, p_ref, w_ref, b_ref, v_ref, np_ref):
    nopos = jnp.maximum(x0_ref[...] + g_ref[...], 0.0)
    np_ref[...] = nopos
    xin = jnp.concatenate([nopos, p_ref[...]], axis=1)
    out = jnp.dot(xin, w_ref[...], preferred_element_type=jnp.float32)
    v_ref[...] = p_ref[...] + jnp.tanh(out + b_ref[...])


def _tc_offset(xw0, agg, pos8, w, b):
    return pl.pallas_call(
        _offset_body,
        grid=(NROW,),
        in_specs=[pl.BlockSpec((RB, 128), lambda i: (i, 0)),
                  pl.BlockSpec((RB, 128), lambda i: (i, 0)),
                  pl.BlockSpec((RB, 8), lambda i: (i, 0)),
                  pl.BlockSpec((136, 8), lambda i: (0, 0)),
                  pl.BlockSpec((1, 8), lambda i: (0, 0))],
        out_specs=[pl.BlockSpec((RB, 8), lambda i: (i, 0)),
                   pl.BlockSpec((RB, 128), lambda i: (i, 0))],
        out_shape=[jax.ShapeDtypeStruct((VP, 8), jnp.float32),
                   jax.ShapeDtypeStruct((VP, 128), jnp.float32)],
    )(xw0, agg, pos8, w, b)


def _sc_mesh():
    return plsc.VectorSubcoreMesh(core_axis_name="c", subcore_axis_name="s",
                                  num_cores=2, num_subcores=16)


def _sc_taps_body(table_hbm, idx_hbm, out_hbm, idx_v, rows_v, sem):
    cid = lax.axis_index("c")
    sid = lax.axis_index("s")
    wid = sid * 2 + cid
    base = wid * TAP_ROWS_PER_TILE
    pltpu.sync_copy(idx_hbm.at[pl.ds(wid * TAP_BATCHES, TAP_BATCHES)], idx_v)

    def body(j, _):
        slot = lax.rem(j, 2)
        cp = pltpu.async_copy(
            table_hbm.at[idx_v.at[j]], rows_v.at[slot], sem)
        cp.wait()
        pltpu.sync_copy(rows_v.at[slot],
                        out_hbm.at[pl.ds(base + j * EB, EB)])
        return 0

    lax.fori_loop(0, TAP_BATCHES, body, 0, unroll=False)


def _sc_gather_taps(img_proj, tap_idx):
    kfn = pl.kernel(
        _sc_taps_body,
        out_type=jax.ShapeDtypeStruct((TAPS_TOT, 128), jnp.float32),
        mesh=_sc_mesh(),
        scratch_types=[
            pltpu.VMEM((TAP_BATCHES, EB), jnp.int32),
            pltpu.VMEM((2, EB, 128), jnp.float32),
            pltpu.SemaphoreType.DMA,
        ],
        compiler_params=pltpu.CompilerParams(use_tc_tiling_on_sc=False),
    )
    return kfn(img_proj, tap_idx.reshape(32 * TAP_BATCHES, EB))


def _sc_agg_body(vw1r_hbm, gidx_hbm, sidx_hbm, out_hbm,
                 gbuf, ibuf, sbuf, rows_v, zbuf, acc, sem):
    cid = lax.axis_index("c")
    sid = lax.axis_index("s")
    ebase = sid * E_PER_TILE
    sbase = sid * (E_PER_TILE // EB)
    KPB = EBATCH // EB

    def zb(i, _):
        zbuf[i, :] = jnp.zeros((16,), jnp.float32)
        return 0
    lax.fori_loop(0, ZCHUNK, zb, 0, unroll=False)

    def do_chunk(f):
        for z in range(VPT // ZCHUNK):
            pltpu.sync_copy(zbuf, acc.at[pl.ds(sid * VPT + z * ZCHUNK,
                                               ZCHUNK)])
        plsc.subcore_barrier()

        def batch(j, _):
            slot = lax.rem(j, 2)
            pltpu.sync_copy(gidx_hbm.at[pl.ds(ebase + j * EBATCH, EBATCH)],
                            gbuf.at[slot])
            pltpu.sync_copy(sidx_hbm.at[pl.ds(sbase + j * KPB, KPB)],
                            sbuf.at[slot])
            for k in range(KPB):
                def mk(i, _, k=k):
                    g = gbuf[slot, pl.ds(k * EB + i * 16, 16)]
                    ibuf[slot, k, pl.ds(i * 16, 16)] = g * NCHUNKS + f
                    return 0
                lax.fori_loop(0, EB // 16, mk, 0, unroll=False)
            for k in range(KPB):
                cp = pltpu.async_copy(
                    vw1r_hbm.at[ibuf.at[slot, k]],
                    rows_v.at[slot, pl.ds(k * EB, EB)], sem)
                cp.wait()
                pltpu.sync_copy(
                    rows_v.at[slot, pl.ds(k * EB, EB)],
                    acc.at[sbuf.at[slot, k]],
                    add=True)
            return 0

        lax.fori_loop(0, NBATCH, batch, 0, unroll=False)
        plsc.subcore_barrier()
        pltpu.sync_copy(
            acc.at[pl.ds(sid * VPT, VPT)],
            out_hbm.at[pl.ds(sid * VPT, VPT), pl.ds(f * 16, 16)])
        plsc.subcore_barrier()

    for fi in range(NCHUNKS // 2):
        do_chunk(cid * (NCHUNKS // 2) + fi)


def _sc_edge_agg(vw1, gidx, sidx2d):
    kfn = pl.kernel(
        _sc_agg_body,
        out_type=jax.ShapeDtypeStruct((VP, 128), jnp.float32),
        mesh=_sc_mesh(),
        scratch_types=[
            pltpu.VMEM((2, EBATCH), jnp.int32),
            pltpu.VMEM((2, EBATCH // EB, EB), jnp.int32),
            pltpu.VMEM((2, EBATCH // EB, EB), jnp.int32),
            pltpu.VMEM((2, EBATCH, 16), jnp.float32),
            pltpu.VMEM((ZCHUNK, 16), jnp.float32),
            pltpu.VMEM_SHARED((ACC_ROWS, 16), jnp.float32),
            pltpu.SemaphoreType.DMA,
        ],
        compiler_params=pltpu.CompilerParams(use_tc_tiling_on_sc=False),
    )
    return kfn(vw1.reshape(VP * NCHUNKS, 16), gidx, sidx2d)


def _pack_gconv_w(p, first_with_feats):
    w0W, w0b = p['w0']
    w1W, w1b = p['w1']
    w0t, w1t = w0W.T, w1W.T
    if first_with_feats:
        def arrange(wt):
            return jnp.concatenate([
                wt[:HIDDEN], wt[HIDDEN:HIDDEN + 3],
                jnp.zeros((5, HIDDEN), jnp.float32),
                wt[HIDDEN + 3:]], axis=0)
    else:
        def arrange(wt):
            return jnp.concatenate([
                wt[:HIDDEN], wt[HIDDEN:HIDDEN + 3],
                jnp.zeros((5, HIDDEN), jnp.float32)], axis=0)
    w = jnp.concatenate([arrange(w0t), arrange(w1t)], axis=1)
    b = jnp.concatenate([w0b, w1b]).reshape(1, 256)
    return w, b


def _pack_offset_w(p):
    oW, ob = p['vert_offset']
    ot = oW.T
    w = jnp.concatenate([ot[:HIDDEN], ot[HIDDEN:HIDDEN + 3],
                         jnp.zeros((5, 3), jnp.float32)], axis=0)
    w = jnp.concatenate([w, jnp.zeros((136, 5), jnp.float32)], axis=1)
    b = jnp.concatenate([ob, jnp.zeros((5,), jnp.float32)]).reshape(1, 8)
    return w, b


def kernel(img_feats, verts_padded, edges_packed, params):
    f32 = jnp.float32
    img_t = jnp.transpose(img_feats, (0, 2, 3, 1)).reshape(N * H_IMG * W_IMG,
                                                           C_IMG)
    bws = [params['stages'][s]['bottleneck'][0].T for s in range(NUM_STAGES)]
    img_proj_all = _tc_imgproj(img_t, jnp.concatenate(bws, axis=1))
    img_projs = [img_proj_all[:, s * 128:(s + 1) * 128] for s in
                 range(NUM_STAGES)]

    src = edges_packed[:, 0]
    dst = edges_packed[:, 1]
    gidx = jnp.concatenate([dst, src])
    sidx = jnp.concatenate([src, dst])
    gidx = jnp.concatenate([gidx, jnp.zeros((E2P - E2,), jnp.int32)])
    sidx = jnp.concatenate(
        [sidx, jnp.full((E2P - E2,), DUMMY_ROW, jnp.int32)])
    sidx2d = sidx.reshape(E2P // EB, EB)

    verts_flat = jnp.pad(verts_padded.reshape(VTOT, 3),
                         ((0, VP - VTOT), (0, 0)))
    pos8 = jnp.pad(verts_flat, ((0, 0), (0, 5)))

    outs = []
    vfeat = None
    for s in range(NUM_STAGES):
        sp = params['stages'][s]
        xs = pos8[:, 0].reshape(784, 128)
        ys = pos8[:, 1].reshape(784, 128)
        tap_idx, tap_w = _tc_prep(xs, ys)
        taps_flat = _sc_gather_taps(img_projs[s], tap_idx.reshape(4 * VP))
        w8 = jnp.pad(jnp.transpose(tap_w.reshape(4, VP)), ((0, 0), (0, 4)))
        bb = sp['bottleneck'][1].reshape(1, 128)
        va = _tc_va(taps_flat, w8, bb)

        if s == 0:
            w, b = _pack_gconv_w(sp['gconvs'][0], False)
            xw0, vw1 = _tc_gconv_first_s0(va, pos8, w, b)
        else:
            w, b = _pack_gconv_w(sp['gconvs'][0], True)
            xw0, vw1 = _tc_gconv_first(va, pos8, vfeat, w, b)
        agg = _sc_edge_agg(vw1, gidx, sidx2d)
        for gi in range(1, STAGE_DEPTH):
            w, b = _pack_gconv_w(sp['gconvs'][gi], False)
            xw0, vw1 = _tc_gconv_mid(xw0, agg, pos8, w, b)
            agg = _sc_edge_agg(vw1, gidx, sidx2d)

        ow, ob = _pack_offset_w(sp)
        verts8, vfeat = _tc_offset(xw0, agg, pos8, ow, ob)
        pos8 = verts8
        outs.append(verts8[:VTOT, :3].reshape(N, V, 3))

    return jnp.stack(outs, axis=0).astype(f32)


# speedup vs baseline: 2.7936x; 1.3938x over previous
import functools

import jax
import jax.numpy as jnp
from jax import lax
from jax.experimental import pallas as pl
from jax.experimental.pallas import tpu as pltpu
from jax.experimental.pallas import tpu_sc as plsc

N, V, E = 10, 10000, 300000
C_IMG, H_IMG, W_IMG = 256, 32, 32
HIDDEN = 128
NUM_STAGES = 3
STAGE_DEPTH = 3

VTOT = N * V
VP = 100352
RB = 512
NROW = VP // RB
E2 = 2 * E
SC_TILES = 16
EB = 128
EBATCH = 512
E_PER_TILE = 38912
E2P = SC_TILES * E_PER_TILE
NBATCH = E_PER_TILE // EBATCH
ACC_ROWS = VP
DUMMY_ROW = VTOT
VPT = VP // SC_TILES
ZCHUNK = 448
NCHUNKS = 8
TAPS_TOT = 4 * VP
TAP_ROWS_PER_TILE = TAPS_TOT // 32
TAP_BATCHES = TAP_ROWS_PER_TILE // EB


def _mm_imgproj_body(x_ref, w_ref, o_ref):
    o_ref[...] = jnp.dot(x_ref[...], w_ref[...],
                         preferred_element_type=jnp.float32)


def _tc_imgproj(img_t, w):
    return pl.pallas_call(
        _mm_imgproj_body,
        grid=(10240 // RB,),
        in_specs=[pl.BlockSpec((RB, C_IMG), lambda i: (i, 0)),
                  pl.BlockSpec((C_IMG, 384), lambda i: (0, 0))],
        out_specs=pl.BlockSpec((RB, 384), lambda i: (i, 0)),
        out_shape=jax.ShapeDtypeStruct((10240, 384), jnp.float32),
    )(img_t, w)


def _prep_body(xs_ref, ys_ref, idx_ref, w_ref):
    gx = xs_ref[...]
    gy = ys_ref[...]
    x = (gx + 1.0) * ((W_IMG - 1) / 2.0)
    y = (1.0 - gy) * ((H_IMG - 1) / 2.0)
    x0 = jnp.floor(x)
    y0 = jnp.floor(y)
    wx1 = x - x0
    wx0 = 1.0 - wx1
    wy1 = y - y0
    wy0 = 1.0 - wy1
    vid = lax.broadcasted_iota(jnp.int32, (784, 128), 0) * 128 + \
        lax.broadcasted_iota(jnp.int32, (784, 128), 1)
    n = jnp.clip(vid // V, 0, N - 1)
    for t, (ix, iy, wt) in enumerate((
            (x0, y0, wx0 * wy0), (x0 + 1.0, y0, wx1 * wy0),
            (x0, y0 + 1.0, wx0 * wy1), (x0 + 1.0, y0 + 1.0, wx1 * wy1))):
        valid = ((ix >= 0.0) & (ix <= W_IMG - 1.0)
                 & (iy >= 0.0) & (iy <= H_IMG - 1.0))
        ixc = jnp.clip(ix, 0.0, W_IMG - 1.0).astype(jnp.int32)
        iyc = jnp.clip(iy, 0.0, H_IMG - 1.0).astype(jnp.int32)
        idx_ref[t] = n * (H_IMG * W_IMG) + iyc * W_IMG + ixc
        w_ref[t] = jnp.where(valid, wt, 0.0)


def _tc_prep(xs, ys):
    return pl.pallas_call(
        _prep_body,
        in_specs=[pl.BlockSpec((784, 128), lambda: (0, 0)),
                  pl.BlockSpec((784, 128), lambda: (0, 0))],
        out_specs=[pl.BlockSpec((4, 784, 128), lambda: (0, 0, 0)),
                   pl.BlockSpec((4, 784, 128), lambda: (0, 0, 0))],
        out_shape=[jax.ShapeDtypeStruct((4, 784, 128), jnp.int32),
                   jax.ShapeDtypeStruct((4, 784, 128), jnp.float32)],
    )(xs, ys)


def _va_body(t0, t1, t2, t3, w_ref, b_ref, o_ref):
    w = w_ref[...]
    acc = t0[...] * w[:, 0:1] + t1[...] * w[:, 1:2]
    acc += t2[...] * w[:, 2:3] + t3[...] * w[:, 3:4]
    o_ref[...] = jnp.maximum(acc + b_ref[...], 0.0)


def _tc_va(taps_flat, w8, bb):
    specs = [pl.BlockSpec((RB, 128), functools.partial(
        lambda i, t: (t * NROW + i, 0), t=t)) for t in range(4)]
    return pl.pallas_call(
        _va_body,
        grid=(NROW,),
        in_specs=specs + [pl.BlockSpec((RB, 8), lambda i: (i, 0)),
                          pl.BlockSpec((1, 128), lambda i: (0, 0))],
        out_specs=pl.BlockSpec((RB, 128), lambda i: (i, 0)),
        out_shape=jax.ShapeDtypeStruct((VP, 128), jnp.float32),
    )(taps_flat, taps_flat, taps_flat, taps_flat, w8, bb)


def _gconv_first_body(a_ref, p_ref, f_ref, w_ref, b_ref, o0_ref, o1_ref):
    xin = jnp.concatenate([a_ref[...], p_ref[...], f_ref[...]], axis=1)
    out = jnp.dot(xin, w_ref[...], preferred_element_type=jnp.float32)
    out += b_ref[...]
    o0_ref[...] = out[:, :HIDDEN]
    o1_ref[...] = out[:, HIDDEN:]


def _tc_gconv_first(va, pos8, vfeat, w, b):
    return pl.pallas_call(
        _gconv_first_body,
        grid=(NROW,),
        in_specs=[pl.BlockSpec((RB, 128), lambda i: (i, 0)),
                  pl.BlockSpec((RB, 8), lambda i: (i, 0)),
                  pl.BlockSpec((RB, 128), lambda i: (i, 0)),
                  pl.BlockSpec((264, 256), lambda i: (0, 0)),
                  pl.BlockSpec((1, 256), lambda i: (0, 0))],
        out_specs=[pl.BlockSpec((RB, 128), lambda i: (i, 0)),
                   pl.BlockSpec((RB, 128), lambda i: (i, 0))],
        out_shape=[jax.ShapeDtypeStruct((VP, 128), jnp.float32),
                   jax.ShapeDtypeStruct((VP, 128), jnp.float32)],
    )(va, pos8, vfeat, w, b)


def _gconv_first_s0_body(a_ref, p_ref, w_ref, b_ref, o0_ref, o1_ref):
    xin = jnp.concatenate([a_ref[...], p_ref[...]], axis=1)
    out = jnp.dot(xin, w_ref[...], preferred_element_type=jnp.float32)
    out += b_ref[...]
    o0_ref[...] = out[:, :HIDDEN]
    o1_ref[...] = out[:, HIDDEN:]


def _tc_gconv_first_s0(va, pos8, w, b):
    return pl.pallas_call(
        _gconv_first_s0_body,
        grid=(NROW,),
        in_specs=[pl.BlockSpec((RB, 128), lambda i: (i, 0)),
                  pl.BlockSpec((RB, 8), lambda i: (i, 0)),
                  pl.BlockSpec((136, 256), lambda i: (0, 0)),
                  pl.BlockSpec((1, 256), lambda i: (0, 0))],
        out_specs=[pl.BlockSpec((RB, 128), lambda i: (i, 0)),
                   pl.BlockSpec((RB, 128), lambda i: (i, 0))],
        out_shape=[jax.ShapeDtypeStruct((VP, 128), jnp.float32),
                   jax.ShapeDtypeStruct((VP, 128), jnp.float32)],
    )(va, pos8, w, b)


def _gconv_mid_body(x0_ref, g_ref, p_ref, w_ref, b_ref, o0_ref, o1_ref):
    nopos = jnp.maximum(x0_ref[...] + g_ref[...], 0.0)
    xin = jnp.concatenate([nopos, p_ref[...]], axis=1)
    out = jnp.dot(xin, w_ref[...], preferred_element_type=jnp.float32)
    out += b_ref[...]
    o0_ref[...] = out[:, :HIDDEN]
    o1_ref[...] = out[:, HIDDEN:]


def _tc_gconv_mid(xw0, agg, pos8, w, b):
    return pl.pallas_call(
        _gconv_mid_body,
        grid=(NROW,),
        in_specs=[pl.BlockSpec((RB, 128), lambda i: (i, 0)),
                  pl.BlockSpec((RB, 128), lambda i: (i, 0)),
                  pl.BlockSpec((RB, 8), lambda i: (i, 0)),
                  pl.BlockSpec((136, 256), lambda i: (0, 0)),
                  pl.BlockSpec((1, 256), lambda i: (0, 0))],
        out_specs=[pl.BlockSpec((RB, 128), lambda i: (i, 0)),
                   pl.BlockSpec((RB, 128), lambda i: (i, 0))],
        out_shape=[jax.ShapeDtypeStruct((VP, 128), jnp.float32),
                   jax.ShapeDtypeStruct((VP, 128), jnp.float32)],
    )(xw0, agg, pos8, w, b)


def _offset_body(x0_ref, g_ref, p_ref, w_ref, b_ref, v_ref, np_ref):
    nopos = jnp.maximum(x0_ref[...] + g_ref[...], 0.0)
    np_ref[...] = nopos
    xin = jnp.concatenate([nopos, p_ref[...]], axis=1)
    out = jnp.dot(xin, w_ref[...], preferred_element_type=jnp.float32)
    v_ref[...] = p_ref[...] + jnp.tanh(out + b_ref[...])


def _tc_offset(xw0, agg, pos8, w, b):
    return pl.pallas_call(
        _offset_body,
        grid=(NROW,),
        in_specs=[pl.BlockSpec((RB, 128), lambda i: (i, 0)),
                  pl.BlockSpec((RB, 128), lambda i: (i, 0)),
                  pl.BlockSpec((RB, 8), lambda i: (i, 0)),
                  pl.BlockSpec((136, 8), lambda i: (0, 0)),
                  pl.BlockSpec((1, 8), lambda i: (0, 0))],
        out_specs=[pl.BlockSpec((RB, 8), lambda i: (i, 0)),
                   pl.BlockSpec((RB, 128), lambda i: (i, 0))],
        out_shape=[jax.ShapeDtypeStruct((VP, 8), jnp.float32),
                   jax.ShapeDtypeStruct((VP, 128), jnp.float32)],
    )(xw0, agg, pos8, w, b)


def _sc_mesh():
    return plsc.VectorSubcoreMesh(core_axis_name="c", subcore_axis_name="s",
                                  num_cores=2, num_subcores=16)


def _sc_taps_body(table_hbm, idx_hbm, out_hbm, idx_v, rows_v, sem_g, sem_w):
    cid = lax.axis_index("c")
    sid = lax.axis_index("s")
    wid = sid * 2 + cid
    base = wid * TAP_ROWS_PER_TILE
    pltpu.sync_copy(idx_hbm.at[pl.ds(wid * TAP_BATCHES, TAP_BATCHES)], idx_v)

    def fire_gather(j, slot):
        pltpu.async_copy(table_hbm.at[idx_v.at[j]], rows_v.at[slot], sem_g)

    def body(j, _):
        slot = lax.rem(j, 2)
        nslot = lax.rem(j + 1, 2)

        @pl.when(j >= 1)
        def _():
            pltpu.make_async_copy(
                rows_v.at[nslot],
                out_hbm.at[pl.ds(base + (j - 1) * EB, EB)], sem_w).wait()
        @pl.when(j + 1 < TAP_BATCHES)
        def _():
            fire_gather(j + 1, nslot)
        pltpu.make_async_copy(table_hbm.at[idx_v.at[j]], rows_v.at[slot],
                              sem_g).wait()
        pltpu.async_copy(rows_v.at[slot],
                         out_hbm.at[pl.ds(base + j * EB, EB)], sem_w)
        return 0

    fire_gather(0, 0)
    lax.fori_loop(0, TAP_BATCHES, body, 0, unroll=False)
    pltpu.make_async_copy(
        rows_v.at[(TAP_BATCHES - 1) % 2],
        out_hbm.at[pl.ds(base + (TAP_BATCHES - 1) * EB, EB)], sem_w).wait()


def _sc_gather_taps(img_proj, tap_idx):
    kfn = pl.kernel(
        _sc_taps_body,
        out_type=jax.ShapeDtypeStruct((TAPS_TOT, 128), jnp.float32),
        mesh=_sc_mesh(),
        scratch_types=[
            pltpu.VMEM((TAP_BATCHES, EB), jnp.int32),
            pltpu.VMEM((2, EB, 128), jnp.float32),
            pltpu.SemaphoreType.DMA,
            pltpu.SemaphoreType.DMA,
        ],
        compiler_params=pltpu.CompilerParams(use_tc_tiling_on_sc=False),
    )
    return kfn(img_proj, tap_idx.reshape(32 * TAP_BATCHES, EB))


def _sc_agg_body(vw1r_hbm, gidx_hbm, sidx_hbm, out_hbm,
                 gbuf, ibuf, sbuf, rows_v, zbuf, acc,
                 sem_i, sem_g, sem_s, sem_z):
    cid = lax.axis_index("c")
    sid = lax.axis_index("s")
    ebase = sid * E_PER_TILE
    sbase = sid * (E_PER_TILE // EB)
    KPB = EBATCH // EB

    def zb(i, _):
        zbuf[i, :] = jnp.zeros((16,), jnp.float32)
        return 0
    lax.fori_loop(0, ZCHUNK, zb, 0, unroll=False)

    def fire_idx(j, slot, slot3):
        pltpu.async_copy(gidx_hbm.at[pl.ds(ebase + j * EBATCH, EBATCH)],
                         gbuf.at[slot], sem_i)
        pltpu.async_copy(sidx_hbm.at[pl.ds(sbase + j * KPB, KPB)],
                         sbuf.at[slot3], sem_i)

    def drain_idx(slot, slot3):
        pltpu.make_async_copy(gidx_hbm.at[pl.ds(0, EBATCH)],
                              gbuf.at[slot], sem_i).wait()
        pltpu.make_async_copy(sidx_hbm.at[pl.ds(0, KPB)],
                              sbuf.at[slot3], sem_i).wait()

    def drain_scatters(slot, slot3):
        for k in range(KPB):
            pltpu.make_async_copy(
                rows_v.at[slot, pl.ds(k * EB, EB)],
                acc.at[sbuf.at[slot3, k]], sem_s).wait()

    def do_chunk(f):
        for z in range(VPT // ZCHUNK):
            pltpu.async_copy(
                zbuf, acc.at[pl.ds(sid * VPT + z * ZCHUNK, ZCHUNK)], sem_z)
        for z in range(VPT // ZCHUNK):
            pltpu.make_async_copy(
                zbuf, acc.at[pl.ds(sid * VPT + z * ZCHUNK, ZCHUNK)],
                sem_z).wait()
        plsc.subcore_barrier()

        def batch(j, _):
            slot = lax.rem(j, 2)
            nslot = lax.rem(j + 1, 2)
            slot3 = lax.rem(j, 3)

            @pl.when(j >= 2)
            def _():
                drain_scatters(slot, lax.rem(j + 1, 3))
            @pl.when(j + 1 < NBATCH)
            def _():
                fire_idx(j + 1, nslot, lax.rem(j + 1, 3))
            drain_idx(slot, slot3)
            for k in range(KPB):
                for i in range(EB // 16):
                    g = gbuf[slot, pl.ds(k * EB + i * 16, 16)]
                    ibuf[slot, k, pl.ds(i * 16, 16)] = g * NCHUNKS + f
            for k in range(KPB):
                pltpu.async_copy(
                    vw1r_hbm.at[ibuf.at[slot, k]],
                    rows_v.at[slot, pl.ds(k * EB, EB)], sem_g)
            for k in range(KPB):
                pltpu.make_async_copy(
                    vw1r_hbm.at[ibuf.at[slot, k]],
                    rows_v.at[slot, pl.ds(k * EB, EB)], sem_g).wait()
            for k in range(KPB):
                pltpu.async_copy(
                    rows_v.at[slot, pl.ds(k * EB, EB)],
                    acc.at[sbuf.at[slot3, k]], sem_s, add=True)
            return 0

        fire_idx(0, 0, 0)
        lax.fori_loop(0, NBATCH, batch, 0, unroll=False)
        drain_scatters((NBATCH - 2) % 2, (NBATCH - 2) % 3)
        drain_scatters((NBATCH - 1) % 2, (NBATCH - 1) % 3)
        plsc.subcore_barrier()
        pltpu.sync_copy(
            acc.at[pl.ds(sid * VPT, VPT)],
            out_hbm.at[pl.ds(sid * VPT, VPT), pl.ds(f * 16, 16)])
        plsc.subcore_barrier()

    for fi in range(NCHUNKS // 2):
        do_chunk(cid * (NCHUNKS // 2) + fi)


def _sc_edge_agg(vw1, gidx, sidx2d):
    kfn = pl.kernel(
        _sc_agg_body,
        out_type=jax.ShapeDtypeStruct((VP, 128), jnp.float32),
        mesh=_sc_mesh(),
        scratch_types=[
            pltpu.VMEM((2, EBATCH), jnp.int32),
            pltpu.VMEM((2, EBATCH // EB, EB), jnp.int32),
            pltpu.VMEM((3, EBATCH // EB, EB), jnp.int32),
            pltpu.VMEM((2, EBATCH, 16), jnp.float32),
            pltpu.VMEM((ZCHUNK, 16), jnp.float32),
            pltpu.VMEM_SHARED((ACC_ROWS, 16), jnp.float32),
            pltpu.SemaphoreType.DMA,
            pltpu.SemaphoreType.DMA,
            pltpu.SemaphoreType.DMA,
            pltpu.SemaphoreType.DMA,
        ],
        compiler_params=pltpu.CompilerParams(use_tc_tiling_on_sc=False),
    )
    return kfn(vw1.reshape(VP * NCHUNKS, 16), gidx, sidx2d)


def _pack_gconv_w(p, first_with_feats):
    w0W, w0b = p['w0']
    w1W, w1b = p['w1']
    w0t, w1t = w0W.T, w1W.T
    if first_with_feats:
        def arrange(wt):
            return jnp.concatenate([
                wt[:HIDDEN], wt[HIDDEN:HIDDEN + 3],
                jnp.zeros((5, HIDDEN), jnp.float32),
                wt[HIDDEN + 3:]], axis=0)
    else:
        def arrange(wt):
            return jnp.concatenate([
                wt[:HIDDEN], wt[HIDDEN:HIDDEN + 3],
                jnp.zeros((5, HIDDEN), jnp.float32)], axis=0)
    w = jnp.concatenate([arrange(w0t), arrange(w1t)], axis=1)
    b = jnp.concatenate([w0b, w1b]).reshape(1, 256)
    return w, b


def _pack_offset_w(p):
    oW, ob = p['vert_offset']
    ot = oW.T
    w = jnp.concatenate([ot[:HIDDEN], ot[HIDDEN:HIDDEN + 3],
                         jnp.zeros((5, 3), jnp.float32)], axis=0)
    w = jnp.concatenate([w, jnp.zeros((136, 5), jnp.float32)], axis=1)
    b = jnp.concatenate([ob, jnp.zeros((5,), jnp.float32)]).reshape(1, 8)
    return w, b


def kernel(img_feats, verts_padded, edges_packed, params):
    f32 = jnp.float32
    img_t = jnp.transpose(img_feats, (0, 2, 3, 1)).reshape(N * H_IMG * W_IMG,
                                                           C_IMG)
    bws = [params['stages'][s]['bottleneck'][0].T for s in range(NUM_STAGES)]
    img_proj_all = _tc_imgproj(img_t, jnp.concatenate(bws, axis=1))
    img_projs = [img_proj_all[:, s * 128:(s + 1) * 128] for s in
                 range(NUM_STAGES)]

    src = edges_packed[:, 0]
    dst = edges_packed[:, 1]
    gidx = jnp.concatenate([dst, src])
    sidx = jnp.concatenate([src, dst])
    gidx = jnp.concatenate([gidx, jnp.zeros((E2P - E2,), jnp.int32)])
    sidx = jnp.concatenate(
        [sidx, jnp.full((E2P - E2,), DUMMY_ROW, jnp.int32)])
    sidx2d = sidx.reshape(E2P // EB, EB)

    verts_flat = jnp.pad(verts_padded.reshape(VTOT, 3),
                         ((0, VP - VTOT), (0, 0)))
    pos8 = jnp.pad(verts_flat, ((0, 0), (0, 5)))

    outs = []
    vfeat = None
    for s in range(NUM_STAGES):
        sp = params['stages'][s]
        xs = pos8[:, 0].reshape(784, 128)
        ys = pos8[:, 1].reshape(784, 128)
        tap_idx, tap_w = _tc_prep(xs, ys)
        taps_flat = _sc_gather_taps(img_projs[s], tap_idx.reshape(4 * VP))
        w8 = jnp.pad(jnp.transpose(tap_w.reshape(4, VP)), ((0, 0), (0, 4)))
        bb = sp['bottleneck'][1].reshape(1, 128)
        va = _tc_va(taps_flat, w8, bb)

        if s == 0:
            w, b = _pack_gconv_w(sp['gconvs'][0], False)
            xw0, vw1 = _tc_gconv_first_s0(va, pos8, w, b)
        else:
            w, b = _pack_gconv_w(sp['gconvs'][0], True)
            xw0, vw1 = _tc_gconv_first(va, pos8, vfeat, w, b)
        agg = _sc_edge_agg(vw1, gidx, sidx2d)
        for gi in range(1, STAGE_DEPTH):
            w, b = _pack_gconv_w(sp['gconvs'][gi], False)
            xw0, vw1 = _tc_gconv_mid(xw0, agg, pos8, w, b)
            agg = _sc_edge_agg(vw1, gidx, sidx2d)

        ow, ob = _pack_offset_w(sp)
        verts8, vfeat = _tc_offset(xw0, agg, pos8, ow, ob)
        pos8 = verts8
        outs.append(verts8[:VTOT, :3].reshape(N, V, 3))

    return jnp.stack(outs, axis=0).astype(f32)


# speedup vs baseline: 2.8567x; 1.0226x over previous
import functools

import jax
import jax.numpy as jnp
from jax import lax
from jax.experimental import pallas as pl
from jax.experimental.pallas import tpu as pltpu
from jax.experimental.pallas import tpu_sc as plsc

N, V, E = 10, 10000, 300000
C_IMG, H_IMG, W_IMG = 256, 32, 32
HIDDEN = 128
NUM_STAGES = 3
STAGE_DEPTH = 3

VTOT = N * V
VP = 100352
RB = 512
NROW = VP // RB
E2 = 2 * E
SC_TILES = 16
EB = 128
EBATCH = 256
E_PER_TILE = 38912
E2P = SC_TILES * E_PER_TILE
NBATCH = E_PER_TILE // EBATCH
ACC_ROWS = VP
DUMMY_ROW = VTOT
VPT = VP // SC_TILES
ZCHUNK = 224
NCHUNKS = 8
TAPS_TOT = 4 * VP
TAP_ROWS_PER_TILE = TAPS_TOT // 32
TAP_BATCHES = TAP_ROWS_PER_TILE // EB


def _mm_imgproj_body(x_ref, w_ref, o_ref):
    o_ref[...] = jnp.dot(x_ref[...], w_ref[...],
                         preferred_element_type=jnp.float32)


def _tc_imgproj(img_t, w):
    return pl.pallas_call(
        _mm_imgproj_body,
        grid=(10240 // RB,),
        in_specs=[pl.BlockSpec((RB, C_IMG), lambda i: (i, 0)),
                  pl.BlockSpec((C_IMG, 384), lambda i: (0, 0))],
        out_specs=pl.BlockSpec((RB, 384), lambda i: (i, 0)),
        out_shape=jax.ShapeDtypeStruct((10240, 384), jnp.float32),
    )(img_t, w)


def _prep_body(xs_ref, ys_ref, idx_ref, w_ref):
    gx = xs_ref[...]
    gy = ys_ref[...]
    x = (gx + 1.0) * ((W_IMG - 1) / 2.0)
    y = (1.0 - gy) * ((H_IMG - 1) / 2.0)
    x0 = jnp.floor(x)
    y0 = jnp.floor(y)
    wx1 = x - x0
    wx0 = 1.0 - wx1
    wy1 = y - y0
    wy0 = 1.0 - wy1
    vid = lax.broadcasted_iota(jnp.int32, (784, 128), 0) * 128 + \
        lax.broadcasted_iota(jnp.int32, (784, 128), 1)
    n = jnp.clip(vid // V, 0, N - 1)
    for t, (ix, iy, wt) in enumerate((
            (x0, y0, wx0 * wy0), (x0 + 1.0, y0, wx1 * wy0),
            (x0, y0 + 1.0, wx0 * wy1), (x0 + 1.0, y0 + 1.0, wx1 * wy1))):
        valid = ((ix >= 0.0) & (ix <= W_IMG - 1.0)
                 & (iy >= 0.0) & (iy <= H_IMG - 1.0))
        ixc = jnp.clip(ix, 0.0, W_IMG - 1.0).astype(jnp.int32)
        iyc = jnp.clip(iy, 0.0, H_IMG - 1.0).astype(jnp.int32)
        idx_ref[t] = n * (H_IMG * W_IMG) + iyc * W_IMG + ixc
        w_ref[t] = jnp.where(valid, wt, 0.0)


def _tc_prep(xs, ys):
    return pl.pallas_call(
        _prep_body,
        in_specs=[pl.BlockSpec((784, 128), lambda: (0, 0)),
                  pl.BlockSpec((784, 128), lambda: (0, 0))],
        out_specs=[pl.BlockSpec((4, 784, 128), lambda: (0, 0, 0)),
                   pl.BlockSpec((4, 784, 128), lambda: (0, 0, 0))],
        out_shape=[jax.ShapeDtypeStruct((4, 784, 128), jnp.int32),
                   jax.ShapeDtypeStruct((4, 784, 128), jnp.float32)],
    )(xs, ys)


def _va_body(t0, t1, t2, t3, w_ref, b_ref, o_ref):
    w = w_ref[...]
    acc = t0[...] * w[:, 0:1] + t1[...] * w[:, 1:2]
    acc += t2[...] * w[:, 2:3] + t3[...] * w[:, 3:4]
    o_ref[...] = jnp.maximum(acc + b_ref[...], 0.0)


def _tc_va(taps_flat, w8, bb):
    specs = [pl.BlockSpec((RB, 128), functools.partial(
        lambda i, t: (t * NROW + i, 0), t=t)) for t in range(4)]
    return pl.pallas_call(
        _va_body,
        grid=(NROW,),
        in_specs=specs + [pl.BlockSpec((RB, 8), lambda i: (i, 0)),
                          pl.BlockSpec((1, 128), lambda i: (0, 0))],
        out_specs=pl.BlockSpec((RB, 128), lambda i: (i, 0)),
        out_shape=jax.ShapeDtypeStruct((VP, 128), jnp.float32),
    )(taps_flat, taps_flat, taps_flat, taps_flat, w8, bb)


def _gconv_first_body(a_ref, p_ref, f_ref, w_ref, b_ref, o0_ref, o1_ref):
    xin = jnp.concatenate([a_ref[...], p_ref[...], f_ref[...]], axis=1)
    out = jnp.dot(xin, w_ref[...], preferred_element_type=jnp.float32)
    out += b_ref[...]
    o0_ref[...] = out[:, :HIDDEN]
    o1_ref[...] = out[:, HIDDEN:]


def _tc_gconv_first(va, pos8, vfeat, w, b):
    return pl.pallas_call(
        _gconv_first_body,
        grid=(NROW,),
        in_specs=[pl.BlockSpec((RB, 128), lambda i: (i, 0)),
                  pl.BlockSpec((RB, 8), lambda i: (i, 0)),
                  pl.BlockSpec((RB, 128), lambda i: (i, 0)),
                  pl.BlockSpec((264, 256), lambda i: (0, 0)),
                  pl.BlockSpec((1, 256), lambda i: (0, 0))],
        out_specs=[pl.BlockSpec((RB, 128), lambda i: (i, 0)),
                   pl.BlockSpec((RB, 128), lambda i: (i, 0))],
        out_shape=[jax.ShapeDtypeStruct((VP, 128), jnp.float32),
                   jax.ShapeDtypeStruct((VP, 128), jnp.float32)],
    )(va, pos8, vfeat, w, b)


def _gconv_first_s0_body(a_ref, p_ref, w_ref, b_ref, o0_ref, o1_ref):
    xin = jnp.concatenate([a_ref[...], p_ref[...]], axis=1)
    out = jnp.dot(xin, w_ref[...], preferred_element_type=jnp.float32)
    out += b_ref[...]
    o0_ref[...] = out[:, :HIDDEN]
    o1_ref[...] = out[:, HIDDEN:]


def _tc_gconv_first_s0(va, pos8, w, b):
    return pl.pallas_call(
        _gconv_first_s0_body,
        grid=(NROW,),
        in_specs=[pl.BlockSpec((RB, 128), lambda i: (i, 0)),
                  pl.BlockSpec((RB, 8), lambda i: (i, 0)),
                  pl.BlockSpec((136, 256), lambda i: (0, 0)),
                  pl.BlockSpec((1, 256), lambda i: (0, 0))],
        out_specs=[pl.BlockSpec((RB, 128), lambda i: (i, 0)),
                   pl.BlockSpec((RB, 128), lambda i: (i, 0))],
        out_shape=[jax.ShapeDtypeStruct((VP, 128), jnp.float32),
                   jax.ShapeDtypeStruct((VP, 128), jnp.float32)],
    )(va, pos8, w, b)


def _gconv_mid_body(x0_ref, g_ref, p_ref, w_ref, b_ref, o0_ref, o1_ref):
    nopos = jnp.maximum(x0_ref[...] + g_ref[...], 0.0)
    xin = jnp.concatenate([nopos, p_ref[...]], axis=1)
    out = jnp.dot(xin, w_ref[...], preferred_element_type=jnp.float32)
    out += b_ref[...]
    o0_ref[...] = out[:, :HIDDEN]
    o1_ref[...] = out[:, HIDDEN:]


def _tc_gconv_mid(xw0, agg, pos8, w, b):
    return pl.pallas_call(
        _gconv_mid_body,
        grid=(NROW,),
        in_specs=[pl.BlockSpec((RB, 128), lambda i: (i, 0)),
                  pl.BlockSpec((RB, 128), lambda i: (i, 0)),
                  pl.BlockSpec((RB, 8), lambda i: (i, 0)),
                  pl.BlockSpec((136, 256), lambda i: (0, 0)),
                  pl.BlockSpec((1, 256), lambda i: (0, 0))],
        out_specs=[pl.BlockSpec((RB, 128), lambda i: (i, 0)),
                   pl.BlockSpec((RB, 128), lambda i: (i, 0))],
        out_shape=[jax.ShapeDtypeStruct((VP, 128), jnp.float32),
                   jax.ShapeDtypeStruct((VP, 128), jnp.float32)],
    )(xw0, agg, pos8, w, b)


def _offset_body(x0_ref, g_ref, p_ref, w_ref, b_ref, v_ref, np_ref):
    nopos = jnp.maximum(x0_ref[...] + g_ref[...], 0.0)
    np_ref[...] = nopos
    xin = jnp.concatenate([nopos, p_ref[...]], axis=1)
    out = jnp.dot(xin, w_ref[...], preferred_element_type=jnp.float32)
    v_ref[...] = p_ref[...] + jnp.tanh(out + b_ref[...])


def _tc_offset(xw0, agg, pos8, w, b):
    return pl.pallas_call(
        _offset_body,
        grid=(NROW,),
        in_specs=[pl.BlockSpec((RB, 128), lambda i: (i, 0)),
                  pl.BlockSpec((RB, 128), lambda i: (i, 0)),
                  pl.BlockSpec((RB, 8), lambda i: (i, 0)),
                  pl.BlockSpec((136, 8), lambda i: (0, 0)),
                  pl.BlockSpec((1, 8), lambda i: (0, 0))],
        out_specs=[pl.BlockSpec((RB, 8), lambda i: (i, 0)),
                   pl.BlockSpec((RB, 128), lambda i: (i, 0))],
        out_shape=[jax.ShapeDtypeStruct((VP, 8), jnp.float32),
                   jax.ShapeDtypeStruct((VP, 128), jnp.float32)],
    )(xw0, agg, pos8, w, b)


def _sc_mesh():
    return plsc.VectorSubcoreMesh(core_axis_name="c", subcore_axis_name="s",
                                  num_cores=2, num_subcores=16)


def _sc_taps_body(table_hbm, idx_hbm, out_hbm, idx_v, rows_v, sem_g, sem_w):
    cid = lax.axis_index("c")
    sid = lax.axis_index("s")
    wid = sid * 2 + cid
    base = wid * TAP_ROWS_PER_TILE
    pltpu.sync_copy(idx_hbm.at[pl.ds(wid * TAP_BATCHES, TAP_BATCHES)], idx_v)

    NB = TAP_BATCHES
    NS = 5

    def fire_gather(j):
        pltpu.async_copy(table_hbm.at[idx_v.at[j]],
                         rows_v.at[lax.rem(j, NS)], sem_g)

    def drain_write(j):
        pltpu.make_async_copy(rows_v.at[lax.rem(j, NS)],
                              out_hbm.at[pl.ds(base + j * EB, EB)],
                              sem_w).wait()

    def body(j, _):
        @pl.when((j >= 2) & (j - 2 < NB))
        def _():
            drain_write(j - 2)

        @pl.when(j + 3 < NB)
        def _():
            fire_gather(j + 3)

        @pl.when(j < NB)
        def _():
            slot = lax.rem(j, NS)
            pltpu.make_async_copy(table_hbm.at[idx_v.at[j]],
                                  rows_v.at[slot], sem_g).wait()
            pltpu.async_copy(rows_v.at[slot],
                             out_hbm.at[pl.ds(base + j * EB, EB)], sem_w)
        return 0

    for p in range(3):
        fire_gather(p)
    lax.fori_loop(0, NB + 2, body, 0, unroll=False)


def _sc_gather_taps(img_proj, tap_idx):
    kfn = pl.kernel(
        _sc_taps_body,
        out_type=jax.ShapeDtypeStruct((TAPS_TOT, 128), jnp.float32),
        mesh=_sc_mesh(),
        scratch_types=[
            pltpu.VMEM((TAP_BATCHES, EB), jnp.int32),
            pltpu.VMEM((5, EB, 128), jnp.float32),
            pltpu.SemaphoreType.DMA,
            pltpu.SemaphoreType.DMA,
        ],
        compiler_params=pltpu.CompilerParams(use_tc_tiling_on_sc=False),
    )
    return kfn(img_proj, tap_idx.reshape(32 * TAP_BATCHES, EB))


def _sc_agg_body(vw1r_hbm, gidx_hbm, sidx_hbm, out_hbm,
                 gbuf, ibuf, sbuf, rows_v, zbuf, acc,
                 sem_i, sem_g, sem_s, sem_z):
    cid = lax.axis_index("c")
    sid = lax.axis_index("s")
    ebase = sid * E_PER_TILE
    sbase = sid * (E_PER_TILE // EB)
    KPB = EBATCH // EB

    def zb(i, _):
        zbuf[i, :] = jnp.zeros((16,), jnp.float32)
        return 0
    lax.fori_loop(0, ZCHUNK, zb, 0, unroll=False)

    def fire_idx(j):
        pltpu.async_copy(gidx_hbm.at[pl.ds(ebase + j * EBATCH, EBATCH)],
                         gbuf.at[lax.rem(j, 3)], sem_i)
        pltpu.async_copy(sidx_hbm.at[pl.ds(sbase + j * KPB, KPB)],
                         sbuf.at[lax.rem(j, 5)], sem_i)

    def drain_idx(j):
        pltpu.make_async_copy(gidx_hbm.at[pl.ds(0, EBATCH)],
                              gbuf.at[lax.rem(j, 3)], sem_i).wait()
        pltpu.make_async_copy(sidx_hbm.at[pl.ds(0, KPB)],
                              sbuf.at[lax.rem(j, 5)], sem_i).wait()

    def fire_gathers(j):
        for k in range(KPB):
            pltpu.async_copy(
                vw1r_hbm.at[ibuf.at[lax.rem(j, 3), k]],
                rows_v.at[lax.rem(j, 4), pl.ds(k * EB, EB)], sem_g)

    def drain_gathers(j):
        for k in range(KPB):
            pltpu.make_async_copy(
                vw1r_hbm.at[ibuf.at[lax.rem(j, 3), k]],
                rows_v.at[lax.rem(j, 4), pl.ds(k * EB, EB)], sem_g).wait()

    def fire_scatters(j):
        for k in range(KPB):
            pltpu.async_copy(
                rows_v.at[lax.rem(j, 4), pl.ds(k * EB, EB)],
                acc.at[sbuf.at[lax.rem(j, 5), k]], sem_s, add=True)

    def drain_scatters(j):
        for k in range(KPB):
            pltpu.make_async_copy(
                rows_v.at[lax.rem(j, 4), pl.ds(k * EB, EB)],
                acc.at[sbuf.at[lax.rem(j, 5), k]], sem_s).wait()

    def do_chunk(f):
        for z in range(VPT // ZCHUNK):
            pltpu.async_copy(
                zbuf, acc.at[pl.ds(sid * VPT + z * ZCHUNK, ZCHUNK)], sem_z)
        for z in range(VPT // ZCHUNK):
            pltpu.make_async_copy(
                zbuf, acc.at[pl.ds(sid * VPT + z * ZCHUNK, ZCHUNK)],
                sem_z).wait()
        plsc.subcore_barrier()

        def batch(j, _):
            @pl.when((j >= 3) & (j - 3 < NBATCH))
            def _():
                drain_scatters(j - 3)

            @pl.when(j + 2 < NBATCH)
            def _():
                fire_idx(j + 2)

            @pl.when(j < NBATCH)
            def _():
                drain_idx(j)
                for k in range(KPB):
                    for i in range(EB // 16):
                        g = gbuf[lax.rem(j, 3), pl.ds(k * EB + i * 16, 16)]
                        ibuf[lax.rem(j, 3), k, pl.ds(i * 16, 16)] = \
                            g * NCHUNKS + f
                fire_gathers(j)

            @pl.when((j >= 2) & (j - 2 < NBATCH))
            def _():
                drain_gathers(j - 2)
                fire_scatters(j - 2)
            return 0

        fire_idx(0)
        fire_idx(1)
        lax.fori_loop(0, NBATCH + 3, batch, 0, unroll=False)
        plsc.subcore_barrier()
        pltpu.sync_copy(
            acc.at[pl.ds(sid * VPT, VPT)],
            out_hbm.at[pl.ds(sid * VPT, VPT), pl.ds(f * 16, 16)])
        plsc.subcore_barrier()

    for fi in range(NCHUNKS // 2):
        do_chunk(cid * (NCHUNKS // 2) + fi)


def _sc_edge_agg(vw1, gidx, sidx2d):
    kfn = pl.kernel(
        _sc_agg_body,
        out_type=jax.ShapeDtypeStruct((VP, 128), jnp.float32),
        mesh=_sc_mesh(),
        scratch_types=[
            pltpu.VMEM((3, EBATCH), jnp.int32),
            pltpu.VMEM((3, EBATCH // EB, EB), jnp.int32),
            pltpu.VMEM((5, EBATCH // EB, EB), jnp.int32),
            pltpu.VMEM((4, EBATCH, 16), jnp.float32),
            pltpu.VMEM((ZCHUNK, 16), jnp.float32),
            pltpu.VMEM_SHARED((ACC_ROWS, 16), jnp.float32),
            pltpu.SemaphoreType.DMA,
            pltpu.SemaphoreType.DMA,
            pltpu.SemaphoreType.DMA,
            pltpu.SemaphoreType.DMA,
        ],
        compiler_params=pltpu.CompilerParams(use_tc_tiling_on_sc=False),
    )
    return kfn(vw1.reshape(VP * NCHUNKS, 16), gidx, sidx2d)


def _pack_gconv_w(p, first_with_feats):
    w0W, w0b = p['w0']
    w1W, w1b = p['w1']
    w0t, w1t = w0W.T, w1W.T
    if first_with_feats:
        def arrange(wt):
            return jnp.concatenate([
                wt[:HIDDEN], wt[HIDDEN:HIDDEN + 3],
                jnp.zeros((5, HIDDEN), jnp.float32),
                wt[HIDDEN + 3:]], axis=0)
    else:
        def arrange(wt):
            return jnp.concatenate([
                wt[:HIDDEN], wt[HIDDEN:HIDDEN + 3],
                jnp.zeros((5, HIDDEN), jnp.float32)], axis=0)
    w = jnp.concatenate([arrange(w0t), arrange(w1t)], axis=1)
    b = jnp.concatenate([w0b, w1b]).reshape(1, 256)
    return w, b


def _pack_offset_w(p):
    oW, ob = p['vert_offset']
    ot = oW.T
    w = jnp.concatenate([ot[:HIDDEN], ot[HIDDEN:HIDDEN + 3],
                         jnp.zeros((5, 3), jnp.float32)], axis=0)
    w = jnp.concatenate([w, jnp.zeros((136, 5), jnp.float32)], axis=1)
    b = jnp.concatenate([ob, jnp.zeros((5,), jnp.float32)]).reshape(1, 8)
    return w, b


def kernel(img_feats, verts_padded, edges_packed, params):
    f32 = jnp.float32
    img_t = jnp.transpose(img_feats, (0, 2, 3, 1)).reshape(N * H_IMG * W_IMG,
                                                           C_IMG)
    bws = [params['stages'][s]['bottleneck'][0].T for s in range(NUM_STAGES)]
    img_proj_all = _tc_imgproj(img_t, jnp.concatenate(bws, axis=1))
    img_projs = [img_proj_all[:, s * 128:(s + 1) * 128] for s in
                 range(NUM_STAGES)]

    src = edges_packed[:, 0]
    dst = edges_packed[:, 1]
    gidx = jnp.concatenate([dst, src])
    sidx = jnp.concatenate([src, dst])
    gidx = jnp.concatenate([gidx, jnp.zeros((E2P - E2,), jnp.int32)])
    sidx = jnp.concatenate(
        [sidx, jnp.full((E2P - E2,), DUMMY_ROW, jnp.int32)])
    sidx2d = sidx.reshape(E2P // EB, EB)

    verts_flat = jnp.pad(verts_padded.reshape(VTOT, 3),
                         ((0, VP - VTOT), (0, 0)))
    pos8 = jnp.pad(verts_flat, ((0, 0), (0, 5)))

    outs = []
    vfeat = None
    for s in range(NUM_STAGES):
        sp = params['stages'][s]
        xs = pos8[:, 0].reshape(784, 128)
        ys = pos8[:, 1].reshape(784, 128)
        tap_idx, tap_w = _tc_prep(xs, ys)
        taps_flat = _sc_gather_taps(img_projs[s], tap_idx.reshape(4 * VP))
        w8 = jnp.pad(jnp.transpose(tap_w.reshape(4, VP)), ((0, 0), (0, 4)))
        bb = sp['bottleneck'][1].reshape(1, 128)
        va = _tc_va(taps_flat, w8, bb)

        if s == 0:
            w, b = _pack_gconv_w(sp['gconvs'][0], False)
            xw0, vw1 = _tc_gconv_first_s0(va, pos8, w, b)
        else:
            w, b = _pack_gconv_w(sp['gconvs'][0], True)
            xw0, vw1 = _tc_gconv_first(va, pos8, vfeat, w, b)
        agg = _sc_edge_agg(vw1, gidx, sidx2d)
        for gi in range(1, STAGE_DEPTH):
            w, b = _pack_gconv_w(sp['gconvs'][gi], False)
            xw0, vw1 = _tc_gconv_mid(xw0, agg, pos8, w, b)
            agg = _sc_edge_agg(vw1, gidx, sidx2d)

        ow, ob = _pack_offset_w(sp)
        verts8, vfeat = _tc_offset(xw0, agg, pos8, ow, ob)
        pos8 = verts8
        outs.append(verts8[:VTOT, :3].reshape(N, V, 3))

    return jnp.stack(outs, axis=0).astype(f32)


# speedup vs baseline: 3.5611x; 1.2466x over previous
import functools

import jax
import jax.numpy as jnp
from jax import lax
from jax.experimental import pallas as pl
from jax.experimental.pallas import tpu as pltpu
from jax.experimental.pallas import tpu_sc as plsc

N, V, E = 10, 10000, 300000
C_IMG, H_IMG, W_IMG = 256, 32, 32
HIDDEN = 128
NUM_STAGES = 3
STAGE_DEPTH = 3

VTOT = N * V
VP = 100352
RB = 512
NROW = VP // RB
E2 = 2 * E
SC_TILES = 16
EB = 128
EBATCH = 256
E_PER_TILE = 38912
E2P = SC_TILES * E_PER_TILE
NBATCH = E_PER_TILE // EBATCH
ACC_ROWS = VP
DUMMY_ROW = VTOT
VPT = VP // SC_TILES
ZCHUNK = 224
NCHUNKS = 8
TAPS_TOT = 4 * VP
TAP_ROWS_PER_TILE = TAPS_TOT // 32
TAP_BATCHES = TAP_ROWS_PER_TILE // EB


def _mm_imgproj_body(x_ref, w_ref, o_ref):
    o_ref[...] = jnp.dot(x_ref[...], w_ref[...],
                         preferred_element_type=jnp.float32
                         ).astype(jnp.bfloat16)


def _tc_imgproj(img_t, w):
    return pl.pallas_call(
        _mm_imgproj_body,
        grid=(10240 // RB,),
        in_specs=[pl.BlockSpec((RB, C_IMG), lambda i: (i, 0)),
                  pl.BlockSpec((C_IMG, 384), lambda i: (0, 0))],
        out_specs=pl.BlockSpec((RB, 384), lambda i: (i, 0)),
        out_shape=jax.ShapeDtypeStruct((10240, 384), jnp.bfloat16),
    )(img_t, w)


def _prep_body(xs_ref, ys_ref, idx_ref, w_ref):
    gx = xs_ref[...]
    gy = ys_ref[...]
    x = (gx + 1.0) * ((W_IMG - 1) / 2.0)
    y = (1.0 - gy) * ((H_IMG - 1) / 2.0)
    x0 = jnp.floor(x)
    y0 = jnp.floor(y)
    wx1 = x - x0
    wx0 = 1.0 - wx1
    wy1 = y - y0
    wy0 = 1.0 - wy1
    vid = lax.broadcasted_iota(jnp.int32, (784, 128), 0) * 128 + \
        lax.broadcasted_iota(jnp.int32, (784, 128), 1)
    n = jnp.clip(vid // V, 0, N - 1)
    for t, (ix, iy, wt) in enumerate((
            (x0, y0, wx0 * wy0), (x0 + 1.0, y0, wx1 * wy0),
            (x0, y0 + 1.0, wx0 * wy1), (x0 + 1.0, y0 + 1.0, wx1 * wy1))):
        valid = ((ix >= 0.0) & (ix <= W_IMG - 1.0)
                 & (iy >= 0.0) & (iy <= H_IMG - 1.0))
        ixc = jnp.clip(ix, 0.0, W_IMG - 1.0).astype(jnp.int32)
        iyc = jnp.clip(iy, 0.0, H_IMG - 1.0).astype(jnp.int32)
        idx_ref[t] = n * (H_IMG * W_IMG) + iyc * W_IMG + ixc
        w_ref[t] = jnp.where(valid, wt, 0.0)


def _tc_prep(xs, ys):
    return pl.pallas_call(
        _prep_body,
        in_specs=[pl.BlockSpec((784, 128), lambda: (0, 0)),
                  pl.BlockSpec((784, 128), lambda: (0, 0))],
        out_specs=[pl.BlockSpec((4, 784, 128), lambda: (0, 0, 0)),
                   pl.BlockSpec((4, 784, 128), lambda: (0, 0, 0))],
        out_shape=[jax.ShapeDtypeStruct((4, 784, 128), jnp.int32),
                   jax.ShapeDtypeStruct((4, 784, 128), jnp.float32)],
    )(xs, ys)


def _va_body(t0, t1, t2, t3, w_ref, b_ref, o_ref):
    w = w_ref[...]
    acc = t0[...].astype(jnp.float32) * w[:, 0:1]
    acc += t1[...].astype(jnp.float32) * w[:, 1:2]
    acc += t2[...].astype(jnp.float32) * w[:, 2:3]
    acc += t3[...].astype(jnp.float32) * w[:, 3:4]
    o_ref[...] = jnp.maximum(acc + b_ref[...], 0.0)


def _tc_va(taps_flat, w8, bb):
    specs = [pl.BlockSpec((RB, 128), functools.partial(
        lambda i, t: (t * NROW + i, 0), t=t)) for t in range(4)]
    return pl.pallas_call(
        _va_body,
        grid=(NROW,),
        in_specs=specs + [pl.BlockSpec((RB, 8), lambda i: (i, 0)),
                          pl.BlockSpec((1, 128), lambda i: (0, 0))],
        out_specs=pl.BlockSpec((RB, 128), lambda i: (i, 0)),
        out_shape=jax.ShapeDtypeStruct((VP, 128), jnp.float32),
    )(taps_flat, taps_flat, taps_flat, taps_flat, w8, bb)


def _gconv_first_body(a_ref, p_ref, f_ref, w_ref, b_ref, o0_ref, o1_ref):
    xin = jnp.concatenate([a_ref[...], p_ref[...], f_ref[...]], axis=1)
    out = jnp.dot(xin, w_ref[...], preferred_element_type=jnp.float32)
    out += b_ref[...]
    o0_ref[...] = out[:, :HIDDEN]
    o1_ref[...] = out[:, HIDDEN:].astype(jnp.bfloat16)


def _tc_gconv_first(va, pos8, vfeat, w, b):
    return pl.pallas_call(
        _gconv_first_body,
        grid=(NROW,),
        in_specs=[pl.BlockSpec((RB, 128), lambda i: (i, 0)),
                  pl.BlockSpec((RB, 8), lambda i: (i, 0)),
                  pl.BlockSpec((RB, 128), lambda i: (i, 0)),
                  pl.BlockSpec((264, 256), lambda i: (0, 0)),
                  pl.BlockSpec((1, 256), lambda i: (0, 0))],
        out_specs=[pl.BlockSpec((RB, 128), lambda i: (i, 0)),
                   pl.BlockSpec((RB, 128), lambda i: (i, 0))],
        out_shape=[jax.ShapeDtypeStruct((VP, 128), jnp.float32),
                   jax.ShapeDtypeStruct((VP, 128), jnp.bfloat16)],
    )(va, pos8, vfeat, w, b)


def _gconv_first_s0_body(a_ref, p_ref, w_ref, b_ref, o0_ref, o1_ref):
    xin = jnp.concatenate([a_ref[...], p_ref[...]], axis=1)
    out = jnp.dot(xin, w_ref[...], preferred_element_type=jnp.float32)
    out += b_ref[...]
    o0_ref[...] = out[:, :HIDDEN]
    o1_ref[...] = out[:, HIDDEN:].astype(jnp.bfloat16)


def _tc_gconv_first_s0(va, pos8, w, b):
    return pl.pallas_call(
        _gconv_first_s0_body,
        grid=(NROW,),
        in_specs=[pl.BlockSpec((RB, 128), lambda i: (i, 0)),
                  pl.BlockSpec((RB, 8), lambda i: (i, 0)),
                  pl.BlockSpec((136, 256), lambda i: (0, 0)),
                  pl.BlockSpec((1, 256), lambda i: (0, 0))],
        out_specs=[pl.BlockSpec((RB, 128), lambda i: (i, 0)),
                   pl.BlockSpec((RB, 128), lambda i: (i, 0))],
        out_shape=[jax.ShapeDtypeStruct((VP, 128), jnp.float32),
                   jax.ShapeDtypeStruct((VP, 128), jnp.bfloat16)],
    )(va, pos8, w, b)


def _gconv_mid_body(x0_ref, g_ref, p_ref, w_ref, b_ref, o0_ref, o1_ref):
    nopos = jnp.maximum(x0_ref[...] + g_ref[...].astype(jnp.float32), 0.0)
    xin = jnp.concatenate([nopos, p_ref[...]], axis=1)
    out = jnp.dot(xin, w_ref[...], preferred_element_type=jnp.float32)
    out += b_ref[...]
    o0_ref[...] = out[:, :HIDDEN]
    o1_ref[...] = out[:, HIDDEN:].astype(jnp.bfloat16)


def _tc_gconv_mid(xw0, agg, pos8, w, b):
    return pl.pallas_call(
        _gconv_mid_body,
        grid=(NROW,),
        in_specs=[pl.BlockSpec((RB, 128), lambda i: (i, 0)),
                  pl.BlockSpec((RB, 128), lambda i: (i, 0)),
                  pl.BlockSpec((RB, 8), lambda i: (i, 0)),
                  pl.BlockSpec((136, 256), lambda i: (0, 0)),
                  pl.BlockSpec((1, 256), lambda i: (0, 0))],
        out_specs=[pl.BlockSpec((RB, 128), lambda i: (i, 0)),
                   pl.BlockSpec((RB, 128), lambda i: (i, 0))],
        out_shape=[jax.ShapeDtypeStruct((VP, 128), jnp.float32),
                   jax.ShapeDtypeStruct((VP, 128), jnp.bfloat16)],
    )(xw0, agg, pos8, w, b)


def _offset_body(x0_ref, g_ref, p_ref, w_ref, b_ref, v_ref, np_ref):
    nopos = jnp.maximum(x0_ref[...] + g_ref[...].astype(jnp.float32), 0.0)
    np_ref[...] = nopos
    xin = jnp.concatenate([nopos, p_ref[...]], axis=1)
    out = jnp.dot(xin, w_ref[...], preferred_element_type=jnp.float32)
    v_ref[...] = p_ref[...] + jnp.tanh(out + b_ref[...])


def _tc_offset(xw0, agg, pos8, w, b):
    return pl.pallas_call(
        _offset_body,
        grid=(NROW,),
        in_specs=[pl.BlockSpec((RB, 128), lambda i: (i, 0)),
                  pl.BlockSpec((RB, 128), lambda i: (i, 0)),
                  pl.BlockSpec((RB, 8), lambda i: (i, 0)),
                  pl.BlockSpec((136, 8), lambda i: (0, 0)),
                  pl.BlockSpec((1, 8), lambda i: (0, 0))],
        out_specs=[pl.BlockSpec((RB, 8), lambda i: (i, 0)),
                   pl.BlockSpec((RB, 128), lambda i: (i, 0))],
        out_shape=[jax.ShapeDtypeStruct((VP, 8), jnp.float32),
                   jax.ShapeDtypeStruct((VP, 128), jnp.float32)],
    )(xw0, agg, pos8, w, b)


def _sc_mesh():
    return plsc.VectorSubcoreMesh(core_axis_name="c", subcore_axis_name="s",
                                  num_cores=2, num_subcores=16)


def _sc_taps_body(table_hbm, idx_hbm, out_hbm, idx_v, rows_v, sem_g, sem_w):
    cid = lax.axis_index("c")
    sid = lax.axis_index("s")
    wid = sid * 2 + cid
    base = wid * TAP_ROWS_PER_TILE
    pltpu.sync_copy(idx_hbm.at[pl.ds(wid * TAP_BATCHES, TAP_BATCHES)], idx_v)

    NB = TAP_BATCHES
    NS = 5

    def fire_gather(j):
        pltpu.async_copy(table_hbm.at[idx_v.at[j]],
                         rows_v.at[lax.rem(j, NS)], sem_g)

    def drain_write(j):
        pltpu.make_async_copy(rows_v.at[lax.rem(j, NS)],
                              out_hbm.at[pl.ds(base + j * EB, EB)],
                              sem_w).wait()

    def body(j, _):
        @pl.when((j >= 2) & (j - 2 < NB))
        def _():
            drain_write(j - 2)

        @pl.when(j + 3 < NB)
        def _():
            fire_gather(j + 3)

        @pl.when(j < NB)
        def _():
            slot = lax.rem(j, NS)
            pltpu.make_async_copy(table_hbm.at[idx_v.at[j]],
                                  rows_v.at[slot], sem_g).wait()
            pltpu.async_copy(rows_v.at[slot],
                             out_hbm.at[pl.ds(base + j * EB, EB)], sem_w)
        return 0

    for p in range(3):
        fire_gather(p)
    lax.fori_loop(0, NB + 2, body, 0, unroll=False)


def _sc_gather_taps(img_proj, tap_idx):
    kfn = pl.kernel(
        _sc_taps_body,
        out_type=jax.ShapeDtypeStruct((TAPS_TOT, 128), jnp.bfloat16),
        mesh=_sc_mesh(),
        scratch_types=[
            pltpu.VMEM((TAP_BATCHES, EB), jnp.int32),
            pltpu.VMEM((5, EB, 128), jnp.bfloat16),
            pltpu.SemaphoreType.DMA,
            pltpu.SemaphoreType.DMA,
        ],
        compiler_params=pltpu.CompilerParams(use_tc_tiling_on_sc=False),
    )
    return kfn(img_proj, tap_idx.reshape(32 * TAP_BATCHES, EB))


def _sc_agg_body(vw1r_hbm, gidx_hbm, sidx_hbm, out_hbm,
                 gbuf, ibuf, sbuf, rows_v, zbuf, acc,
                 sem_i, sem_g, sem_s, sem_z):
    cid = lax.axis_index("c")
    sid = lax.axis_index("s")
    ebase = sid * E_PER_TILE
    sbase = sid * (E_PER_TILE // EB)
    KPB = EBATCH // EB

    def zb(i, _):
        zbuf[i, :] = jnp.zeros((32,), jnp.bfloat16)
        return 0
    lax.fori_loop(0, ZCHUNK, zb, 0, unroll=False)

    def fire_idx(j):
        pltpu.async_copy(gidx_hbm.at[pl.ds(ebase + j * EBATCH, EBATCH)],
                         gbuf.at[lax.rem(j, 3)], sem_i)
        pltpu.async_copy(sidx_hbm.at[pl.ds(sbase + j * KPB, KPB)],
                         sbuf.at[lax.rem(j, 5)], sem_i)

    def drain_idx(j):
        pltpu.make_async_copy(gidx_hbm.at[pl.ds(0, EBATCH)],
                              gbuf.at[lax.rem(j, 3)], sem_i).wait()
        pltpu.make_async_copy(sidx_hbm.at[pl.ds(0, KPB)],
                              sbuf.at[lax.rem(j, 5)], sem_i).wait()

    def fire_gathers(j):
        for k in range(KPB):
            pltpu.async_copy(
                vw1r_hbm.at[ibuf.at[lax.rem(j, 3), k]],
                rows_v.at[lax.rem(j, 4), pl.ds(k * EB, EB)], sem_g)

    def drain_gathers(j):
        for k in range(KPB):
            pltpu.make_async_copy(
                vw1r_hbm.at[ibuf.at[lax.rem(j, 3), k]],
                rows_v.at[lax.rem(j, 4), pl.ds(k * EB, EB)], sem_g).wait()

    def fire_scatters(j):
        for k in range(KPB):
            pltpu.async_copy(
                rows_v.at[lax.rem(j, 4), pl.ds(k * EB, EB)],
                acc.at[sbuf.at[lax.rem(j, 5), k]], sem_s, add=True)

    def drain_scatters(j):
        for k in range(KPB):
            pltpu.make_async_copy(
                rows_v.at[lax.rem(j, 4), pl.ds(k * EB, EB)],
                acc.at[sbuf.at[lax.rem(j, 5), k]], sem_s).wait()

    def do_chunk(f):
        for z in range(VPT // ZCHUNK):
            pltpu.async_copy(
                zbuf, acc.at[pl.ds(sid * VPT + z * ZCHUNK, ZCHUNK)], sem_z)
        for z in range(VPT // ZCHUNK):
            pltpu.make_async_copy(
                zbuf, acc.at[pl.ds(sid * VPT + z * ZCHUNK, ZCHUNK)],
                sem_z).wait()
        plsc.subcore_barrier()

        def batch(j, _):
            @pl.when((j >= 3) & (j - 3 < NBATCH))
            def _():
                drain_scatters(j - 3)

            @pl.when(j + 2 < NBATCH)
            def _():
                fire_idx(j + 2)

            @pl.when(j < NBATCH)
            def _():
                drain_idx(j)
                for k in range(KPB):
                    for i in range(EB // 16):
                        g = gbuf[lax.rem(j, 3), pl.ds(k * EB + i * 16, 16)]
                        ibuf[lax.rem(j, 3), k, pl.ds(i * 16, 16)] = \
                            g * 4 + f
                fire_gathers(j)

            @pl.when((j >= 2) & (j - 2 < NBATCH))
            def _():
                drain_gathers(j - 2)
                fire_scatters(j - 2)
            return 0

        fire_idx(0)
        fire_idx(1)
        lax.fori_loop(0, NBATCH + 3, batch, 0, unroll=False)
        plsc.subcore_barrier()
        pltpu.sync_copy(
            acc.at[pl.ds(sid * VPT, VPT)],
            out_hbm.at[pl.ds(sid * VPT, VPT), pl.ds(f * 32, 32)])
        plsc.subcore_barrier()

    for fi in range(2):
        do_chunk(cid * 2 + fi)


def _sc_edge_agg(vw1, gidx, sidx2d):
    kfn = pl.kernel(
        _sc_agg_body,
        out_type=jax.ShapeDtypeStruct((VP, 128), jnp.bfloat16),
        mesh=_sc_mesh(),
        scratch_types=[
            pltpu.VMEM((3, EBATCH), jnp.int32),
            pltpu.VMEM((3, EBATCH // EB, EB), jnp.int32),
            pltpu.VMEM((5, EBATCH // EB, EB), jnp.int32),
            pltpu.VMEM((4, EBATCH, 32), jnp.bfloat16),
            pltpu.VMEM((ZCHUNK, 32), jnp.bfloat16),
            pltpu.VMEM_SHARED((ACC_ROWS, 32), jnp.bfloat16),
            pltpu.SemaphoreType.DMA,
            pltpu.SemaphoreType.DMA,
            pltpu.SemaphoreType.DMA,
            pltpu.SemaphoreType.DMA,
        ],
        compiler_params=pltpu.CompilerParams(use_tc_tiling_on_sc=False),
    )
    return kfn(vw1.reshape(VP * 4, 32), gidx, sidx2d)


def _pack_gconv_w(p, first_with_feats):
    w0W, w0b = p['w0']
    w1W, w1b = p['w1']
    w0t, w1t = w0W.T, w1W.T
    if first_with_feats:
        def arrange(wt):
            return jnp.concatenate([
                wt[:HIDDEN], wt[HIDDEN:HIDDEN + 3],
                jnp.zeros((5, HIDDEN), jnp.float32),
                wt[HIDDEN + 3:]], axis=0)
    else:
        def arrange(wt):
            return jnp.concatenate([
                wt[:HIDDEN], wt[HIDDEN:HIDDEN + 3],
                jnp.zeros((5, HIDDEN), jnp.float32)], axis=0)
    w = jnp.concatenate([arrange(w0t), arrange(w1t)], axis=1)
    b = jnp.concatenate([w0b, w1b]).reshape(1, 256)
    return w, b


def _pack_offset_w(p):
    oW, ob = p['vert_offset']
    ot = oW.T
    w = jnp.concatenate([ot[:HIDDEN], ot[HIDDEN:HIDDEN + 3],
                         jnp.zeros((5, 3), jnp.float32)], axis=0)
    w = jnp.concatenate([w, jnp.zeros((136, 5), jnp.float32)], axis=1)
    b = jnp.concatenate([ob, jnp.zeros((5,), jnp.float32)]).reshape(1, 8)
    return w, b


def kernel(img_feats, verts_padded, edges_packed, params):
    f32 = jnp.float32
    img_t = jnp.transpose(img_feats, (0, 2, 3, 1)).reshape(N * H_IMG * W_IMG,
                                                           C_IMG)
    bws = [params['stages'][s]['bottleneck'][0].T for s in range(NUM_STAGES)]
    img_proj_all = _tc_imgproj(img_t, jnp.concatenate(bws, axis=1))
    img_projs = [img_proj_all[:, s * 128:(s + 1) * 128] for s in
                 range(NUM_STAGES)]

    src = edges_packed[:, 0]
    dst = edges_packed[:, 1]
    gidx = jnp.concatenate([dst, src])
    sidx = jnp.concatenate([src, dst])
    gidx = jnp.concatenate([gidx, jnp.zeros((E2P - E2,), jnp.int32)])
    sidx = jnp.concatenate(
        [sidx, jnp.full((E2P - E2,), DUMMY_ROW, jnp.int32)])
    sidx2d = sidx.reshape(E2P // EB, EB)

    verts_flat = jnp.pad(verts_padded.reshape(VTOT, 3),
                         ((0, VP - VTOT), (0, 0)))
    pos8 = jnp.pad(verts_flat, ((0, 0), (0, 5)))

    outs = []
    vfeat = None
    for s in range(NUM_STAGES):
        sp = params['stages'][s]
        xs = pos8[:, 0].reshape(784, 128)
        ys = pos8[:, 1].reshape(784, 128)
        tap_idx, tap_w = _tc_prep(xs, ys)
        taps_flat = _sc_gather_taps(img_projs[s], tap_idx.reshape(4 * VP))
        w8 = jnp.pad(jnp.transpose(tap_w.reshape(4, VP)), ((0, 0), (0, 4)))
        bb = sp['bottleneck'][1].reshape(1, 128)
        va = _tc_va(taps_flat, w8, bb)

        if s == 0:
            w, b = _pack_gconv_w(sp['gconvs'][0], False)
            xw0, vw1 = _tc_gconv_first_s0(va, pos8, w, b)
        else:
            w, b = _pack_gconv_w(sp['gconvs'][0], True)
            xw0, vw1 = _tc_gconv_first(va, pos8, vfeat, w, b)
        agg = _sc_edge_agg(vw1, gidx, sidx2d)
        for gi in range(1, STAGE_DEPTH):
            w, b = _pack_gconv_w(sp['gconvs'][gi], False)
            xw0, vw1 = _tc_gconv_mid(xw0, agg, pos8, w, b)
            agg = _sc_edge_agg(vw1, gidx, sidx2d)

        ow, ob = _pack_offset_w(sp)
        verts8, vfeat = _tc_offset(xw0, agg, pos8, ow, ob)
        pos8 = verts8
        outs.append(verts8[:VTOT, :3].reshape(N, V, 3))

    return jnp.stack(outs, axis=0).astype(f32)


# speedup vs baseline: 3.6171x; 1.0157x over previous
import functools

import jax
import jax.numpy as jnp
from jax import lax
from jax.experimental import pallas as pl
from jax.experimental.pallas import tpu as pltpu
from jax.experimental.pallas import tpu_sc as plsc

N, V, E = 10, 10000, 300000
C_IMG, H_IMG, W_IMG = 256, 32, 32
HIDDEN = 128
NUM_STAGES = 3
STAGE_DEPTH = 3

VTOT = N * V
VP = 100352
RB = 512
NROW = VP // RB
E2 = 2 * E
SC_TILES = 16
EB = 128
EBATCH = 256
E_PER_TILE = 38912
E2P = SC_TILES * E_PER_TILE
NBATCH = E_PER_TILE // EBATCH
ACC_ROWS = VP
DUMMY_ROW = VTOT
VPT = VP // SC_TILES
ZCHUNK = 224
NCHUNKS = 8
TAPS_TOT = 4 * VP
TAP_ROWS_PER_TILE = TAPS_TOT // 32
TAP_BATCHES = TAP_ROWS_PER_TILE // EB


def _mm_imgproj_body(x_ref, w_ref, o_ref):
    o_ref[...] = jnp.dot(x_ref[...].astype(jnp.bfloat16), w_ref[...],
                         preferred_element_type=jnp.float32
                         ).astype(jnp.bfloat16)


def _tc_imgproj(img_t, w):
    return pl.pallas_call(
        _mm_imgproj_body,
        grid=(10240 // RB,),
        in_specs=[pl.BlockSpec((RB, C_IMG), lambda i: (i, 0)),
                  pl.BlockSpec((C_IMG, 384), lambda i: (0, 0))],
        out_specs=pl.BlockSpec((RB, 384), lambda i: (i, 0)),
        out_shape=jax.ShapeDtypeStruct((10240, 384), jnp.bfloat16),
    )(img_t, w)


def _prep_body(xs_ref, ys_ref, idx_ref, w_ref):
    gx = xs_ref[...]
    gy = ys_ref[...]
    x = (gx + 1.0) * ((W_IMG - 1) / 2.0)
    y = (1.0 - gy) * ((H_IMG - 1) / 2.0)
    x0 = jnp.floor(x)
    y0 = jnp.floor(y)
    wx1 = x - x0
    wx0 = 1.0 - wx1
    wy1 = y - y0
    wy0 = 1.0 - wy1
    vid = lax.broadcasted_iota(jnp.int32, (784, 128), 0) * 128 + \
        lax.broadcasted_iota(jnp.int32, (784, 128), 1)
    n = jnp.clip(vid // V, 0, N - 1)
    for t, (ix, iy, wt) in enumerate((
            (x0, y0, wx0 * wy0), (x0 + 1.0, y0, wx1 * wy0),
            (x0, y0 + 1.0, wx0 * wy1), (x0 + 1.0, y0 + 1.0, wx1 * wy1))):
        valid = ((ix >= 0.0) & (ix <= W_IMG - 1.0)
                 & (iy >= 0.0) & (iy <= H_IMG - 1.0))
        ixc = jnp.clip(ix, 0.0, W_IMG - 1.0).astype(jnp.int32)
        iyc = jnp.clip(iy, 0.0, H_IMG - 1.0).astype(jnp.int32)
        idx_ref[t] = n * (H_IMG * W_IMG) + iyc * W_IMG + ixc
        w_ref[t] = jnp.where(valid, wt, 0.0)


def _tc_prep(xs, ys):
    return pl.pallas_call(
        _prep_body,
        in_specs=[pl.BlockSpec((784, 128), lambda: (0, 0)),
                  pl.BlockSpec((784, 128), lambda: (0, 0))],
        out_specs=[pl.BlockSpec((4, 784, 128), lambda: (0, 0, 0)),
                   pl.BlockSpec((4, 784, 128), lambda: (0, 0, 0))],
        out_shape=[jax.ShapeDtypeStruct((4, 784, 128), jnp.int32),
                   jax.ShapeDtypeStruct((4, 784, 128), jnp.float32)],
    )(xs, ys)


def _va_body(t0, t1, t2, t3, w_ref, b_ref, o_ref):
    w = w_ref[...]
    acc = t0[...].astype(jnp.float32) * w[:, 0:1]
    acc += t1[...].astype(jnp.float32) * w[:, 1:2]
    acc += t2[...].astype(jnp.float32) * w[:, 2:3]
    acc += t3[...].astype(jnp.float32) * w[:, 3:4]
    o_ref[...] = jnp.maximum(acc + b_ref[...], 0.0).astype(jnp.bfloat16)


def _tc_va(taps_flat, w8, bb):
    specs = [pl.BlockSpec((RB, 128), functools.partial(
        lambda i, t: (t * NROW + i, 0), t=t)) for t in range(4)]
    return pl.pallas_call(
        _va_body,
        grid=(NROW,),
        in_specs=specs + [pl.BlockSpec((RB, 8), lambda i: (i, 0)),
                          pl.BlockSpec((1, 128), lambda i: (0, 0))],
        out_specs=pl.BlockSpec((RB, 128), lambda i: (i, 0)),
        out_shape=jax.ShapeDtypeStruct((VP, 128), jnp.bfloat16),
    )(taps_flat, taps_flat, taps_flat, taps_flat, w8, bb)


def _gconv_first_body(a_ref, p_ref, f_ref, w_ref, b_ref, o0_ref, o1_ref):
    xin = jnp.concatenate([a_ref[...], p_ref[...].astype(jnp.bfloat16),
                           f_ref[...]], axis=1)
    out = jnp.dot(xin, w_ref[...], preferred_element_type=jnp.float32)
    out += b_ref[...]
    o0_ref[...] = out[:, :HIDDEN].astype(jnp.bfloat16)
    o1_ref[...] = out[:, HIDDEN:].astype(jnp.bfloat16)


def _tc_gconv_first(va, pos8, vfeat, w, b):
    return pl.pallas_call(
        _gconv_first_body,
        grid=(NROW,),
        in_specs=[pl.BlockSpec((RB, 128), lambda i: (i, 0)),
                  pl.BlockSpec((RB, 8), lambda i: (i, 0)),
                  pl.BlockSpec((RB, 128), lambda i: (i, 0)),
                  pl.BlockSpec((264, 256), lambda i: (0, 0)),
                  pl.BlockSpec((1, 256), lambda i: (0, 0))],
        out_specs=[pl.BlockSpec((RB, 128), lambda i: (i, 0)),
                   pl.BlockSpec((RB, 128), lambda i: (i, 0))],
        out_shape=[jax.ShapeDtypeStruct((VP, 128), jnp.bfloat16),
                   jax.ShapeDtypeStruct((VP, 128), jnp.bfloat16)],
    )(va, pos8, vfeat, w, b)


def _gconv_first_s0_body(a_ref, p_ref, w_ref, b_ref, o0_ref, o1_ref):
    xin = jnp.concatenate([a_ref[...], p_ref[...].astype(jnp.bfloat16)],
                          axis=1)
    out = jnp.dot(xin, w_ref[...], preferred_element_type=jnp.float32)
    out += b_ref[...]
    o0_ref[...] = out[:, :HIDDEN].astype(jnp.bfloat16)
    o1_ref[...] = out[:, HIDDEN:].astype(jnp.bfloat16)


def _tc_gconv_first_s0(va, pos8, w, b):
    return pl.pallas_call(
        _gconv_first_s0_body,
        grid=(NROW,),
        in_specs=[pl.BlockSpec((RB, 128), lambda i: (i, 0)),
                  pl.BlockSpec((RB, 8), lambda i: (i, 0)),
                  pl.BlockSpec((136, 256), lambda i: (0, 0)),
                  pl.BlockSpec((1, 256), lambda i: (0, 0))],
        out_specs=[pl.BlockSpec((RB, 128), lambda i: (i, 0)),
                   pl.BlockSpec((RB, 128), lambda i: (i, 0))],
        out_shape=[jax.ShapeDtypeStruct((VP, 128), jnp.bfloat16),
                   jax.ShapeDtypeStruct((VP, 128), jnp.bfloat16)],
    )(va, pos8, w, b)


def _gconv_mid_body(x0_ref, g_ref, p_ref, w_ref, b_ref, o0_ref, o1_ref):
    nopos = jnp.maximum(x0_ref[...].astype(jnp.float32)
                        + g_ref[...].astype(jnp.float32), 0.0)
    xin = jnp.concatenate([nopos.astype(jnp.bfloat16),
                           p_ref[...].astype(jnp.bfloat16)], axis=1)
    out = jnp.dot(xin, w_ref[...], preferred_element_type=jnp.float32)
    out += b_ref[...]
    o0_ref[...] = out[:, :HIDDEN].astype(jnp.bfloat16)
    o1_ref[...] = out[:, HIDDEN:].astype(jnp.bfloat16)


def _tc_gconv_mid(xw0, agg, pos8, w, b):
    return pl.pallas_call(
        _gconv_mid_body,
        grid=(NROW,),
        in_specs=[pl.BlockSpec((RB, 128), lambda i: (i, 0)),
                  pl.BlockSpec((RB, 128), lambda i: (i, 0)),
                  pl.BlockSpec((RB, 8), lambda i: (i, 0)),
                  pl.BlockSpec((136, 256), lambda i: (0, 0)),
                  pl.BlockSpec((1, 256), lambda i: (0, 0))],
        out_specs=[pl.BlockSpec((RB, 128), lambda i: (i, 0)),
                   pl.BlockSpec((RB, 128), lambda i: (i, 0))],
        out_shape=[jax.ShapeDtypeStruct((VP, 128), jnp.bfloat16),
                   jax.ShapeDtypeStruct((VP, 128), jnp.bfloat16)],
    )(xw0, agg, pos8, w, b)


def _offset_body(x0_ref, g_ref, p_ref, w_ref, b_ref, v_ref, np_ref):
    nopos = jnp.maximum(x0_ref[...].astype(jnp.float32)
                        + g_ref[...].astype(jnp.float32), 0.0)
    np_ref[...] = nopos.astype(jnp.bfloat16)
    xin = jnp.concatenate([nopos.astype(jnp.bfloat16),
                           p_ref[...].astype(jnp.bfloat16)], axis=1)
    out = jnp.dot(xin, w_ref[...], preferred_element_type=jnp.float32)
    v_ref[...] = p_ref[...] + jnp.tanh(out + b_ref[...])


def _tc_offset(xw0, agg, pos8, w, b):
    return pl.pallas_call(
        _offset_body,
        grid=(NROW,),
        in_specs=[pl.BlockSpec((RB, 128), lambda i: (i, 0)),
                  pl.BlockSpec((RB, 128), lambda i: (i, 0)),
                  pl.BlockSpec((RB, 8), lambda i: (i, 0)),
                  pl.BlockSpec((136, 8), lambda i: (0, 0)),
                  pl.BlockSpec((1, 8), lambda i: (0, 0))],
        out_specs=[pl.BlockSpec((RB, 8), lambda i: (i, 0)),
                   pl.BlockSpec((RB, 128), lambda i: (i, 0))],
        out_shape=[jax.ShapeDtypeStruct((VP, 8), jnp.float32),
                   jax.ShapeDtypeStruct((VP, 128), jnp.bfloat16)],
    )(xw0, agg, pos8, w, b)


def _sc_mesh():
    return plsc.VectorSubcoreMesh(core_axis_name="c", subcore_axis_name="s",
                                  num_cores=2, num_subcores=16)


def _sc_taps_body(table_hbm, idx_hbm, out_hbm, idx_v, rows_v, sem_g, sem_w):
    cid = lax.axis_index("c")
    sid = lax.axis_index("s")
    wid = sid * 2 + cid
    base = wid * TAP_ROWS_PER_TILE
    pltpu.sync_copy(idx_hbm.at[pl.ds(wid * TAP_BATCHES, TAP_BATCHES)], idx_v)

    NB = TAP_BATCHES
    NS = 5

    def fire_gather(j):
        pltpu.async_copy(table_hbm.at[idx_v.at[j]],
                         rows_v.at[lax.rem(j, NS)], sem_g)

    def drain_write(j):
        pltpu.make_async_copy(rows_v.at[lax.rem(j, NS)],
                              out_hbm.at[pl.ds(base + j * EB, EB)],
                              sem_w).wait()

    def body(j, _):
        @pl.when((j >= 2) & (j - 2 < NB))
        def _():
            drain_write(j - 2)

        @pl.when(j + 3 < NB)
        def _():
            fire_gather(j + 3)

        @pl.when(j < NB)
        def _():
            slot = lax.rem(j, NS)
            pltpu.make_async_copy(table_hbm.at[idx_v.at[j]],
                                  rows_v.at[slot], sem_g).wait()
            pltpu.async_copy(rows_v.at[slot],
                             out_hbm.at[pl.ds(base + j * EB, EB)], sem_w)
        return 0

    for p in range(3):
        fire_gather(p)
    lax.fori_loop(0, NB + 2, body, 0, unroll=False)


def _sc_gather_taps(img_proj, tap_idx):
    kfn = pl.kernel(
        _sc_taps_body,
        out_type=jax.ShapeDtypeStruct((TAPS_TOT, 128), jnp.bfloat16),
        mesh=_sc_mesh(),
        scratch_types=[
            pltpu.VMEM((TAP_BATCHES, EB), jnp.int32),
            pltpu.VMEM((5, EB, 128), jnp.bfloat16),
            pltpu.SemaphoreType.DMA,
            pltpu.SemaphoreType.DMA,
        ],
        compiler_params=pltpu.CompilerParams(use_tc_tiling_on_sc=False),
    )
    return kfn(img_proj, tap_idx.reshape(32 * TAP_BATCHES, EB))


def _sc_agg_body(vw1r_hbm, gidx_hbm, sidx_hbm, out_hbm,
                 gbuf, ibuf, sbuf, rows_v, zbuf, acc,
                 sem_i, sem_g, sem_s, sem_z):
    cid = lax.axis_index("c")
    sid = lax.axis_index("s")
    ebase = sid * E_PER_TILE
    sbase = sid * (E_PER_TILE // EB)
    KPB = EBATCH // EB

    def zb(i, _):
        zbuf[i, :] = jnp.zeros((32,), jnp.bfloat16)
        return 0
    lax.fori_loop(0, ZCHUNK, zb, 0, unroll=False)

    def fire_idx(j):
        pltpu.async_copy(gidx_hbm.at[pl.ds(ebase + j * EBATCH, EBATCH)],
                         gbuf.at[lax.rem(j, 3)], sem_i)
        pltpu.async_copy(sidx_hbm.at[pl.ds(sbase + j * KPB, KPB)],
                         sbuf.at[lax.rem(j, 5)], sem_i)

    def drain_idx(j):
        pltpu.make_async_copy(gidx_hbm.at[pl.ds(0, EBATCH)],
                              gbuf.at[lax.rem(j, 3)], sem_i).wait()
        pltpu.make_async_copy(sidx_hbm.at[pl.ds(0, KPB)],
                              sbuf.at[lax.rem(j, 5)], sem_i).wait()

    def fire_gathers(j):
        for k in range(KPB):
            pltpu.async_copy(
                vw1r_hbm.at[ibuf.at[lax.rem(j, 3), k]],
                rows_v.at[lax.rem(j, 4), pl.ds(k * EB, EB)], sem_g)

    def drain_gathers(j):
        for k in range(KPB):
            pltpu.make_async_copy(
                vw1r_hbm.at[ibuf.at[lax.rem(j, 3), k]],
                rows_v.at[lax.rem(j, 4), pl.ds(k * EB, EB)], sem_g).wait()

    def fire_scatters(j):
        for k in range(KPB):
            pltpu.async_copy(
                rows_v.at[lax.rem(j, 4), pl.ds(k * EB, EB)],
                acc.at[sbuf.at[lax.rem(j, 5), k]], sem_s, add=True)

    def drain_scatters(j):
        for k in range(KPB):
            pltpu.make_async_copy(
                rows_v.at[lax.rem(j, 4), pl.ds(k * EB, EB)],
                acc.at[sbuf.at[lax.rem(j, 5), k]], sem_s).wait()

    def do_chunk(f):
        for z in range(VPT // ZCHUNK):
            pltpu.async_copy(
                zbuf, acc.at[pl.ds(sid * VPT + z * ZCHUNK, ZCHUNK)], sem_z)
        for z in range(VPT // ZCHUNK):
            pltpu.make_async_copy(
                zbuf, acc.at[pl.ds(sid * VPT + z * ZCHUNK, ZCHUNK)],
                sem_z).wait()
        plsc.subcore_barrier()

        def batch(j, _):
            @pl.when((j >= 3) & (j - 3 < NBATCH))
            def _():
                drain_scatters(j - 3)

            @pl.when(j + 2 < NBATCH)
            def _():
                fire_idx(j + 2)

            @pl.when(j < NBATCH)
            def _():
                drain_idx(j)
                for k in range(KPB):
                    for i in range(EB // 16):
                        g = gbuf[lax.rem(j, 3), pl.ds(k * EB + i * 16, 16)]
                        ibuf[lax.rem(j, 3), k, pl.ds(i * 16, 16)] = \
                            g * 4 + f
                fire_gathers(j)

            @pl.when((j >= 2) & (j - 2 < NBATCH))
            def _():
                drain_gathers(j - 2)
                fire_scatters(j - 2)
            return 0

        fire_idx(0)
        fire_idx(1)
        lax.fori_loop(0, NBATCH + 3, batch, 0, unroll=False)
        plsc.subcore_barrier()
        pltpu.sync_copy(
            acc.at[pl.ds(sid * VPT, VPT)],
            out_hbm.at[pl.ds(sid * VPT, VPT), pl.ds(f * 32, 32)])
        plsc.subcore_barrier()

    for fi in range(2):
        do_chunk(cid * 2 + fi)


def _sc_edge_agg(vw1, gidx, sidx2d):
    kfn = pl.kernel(
        _sc_agg_body,
        out_type=jax.ShapeDtypeStruct((VP, 128), jnp.bfloat16),
        mesh=_sc_mesh(),
        scratch_types=[
            pltpu.VMEM((3, EBATCH), jnp.int32),
            pltpu.VMEM((3, EBATCH // EB, EB), jnp.int32),
            pltpu.VMEM((5, EBATCH // EB, EB), jnp.int32),
            pltpu.VMEM((4, EBATCH, 32), jnp.bfloat16),
            pltpu.VMEM((ZCHUNK, 32), jnp.bfloat16),
            pltpu.VMEM_SHARED((ACC_ROWS, 32), jnp.bfloat16),
            pltpu.SemaphoreType.DMA,
            pltpu.SemaphoreType.DMA,
            pltpu.SemaphoreType.DMA,
            pltpu.SemaphoreType.DMA,
        ],
        compiler_params=pltpu.CompilerParams(use_tc_tiling_on_sc=False),
    )
    return kfn(vw1.reshape(VP * 4, 32), gidx, sidx2d)


def _pack_gconv_w(p, first_with_feats):
    w0W, w0b = p['w0']
    w1W, w1b = p['w1']
    w0t, w1t = w0W.T, w1W.T
    if first_with_feats:
        def arrange(wt):
            return jnp.concatenate([
                wt[:HIDDEN], wt[HIDDEN:HIDDEN + 3],
                jnp.zeros((5, HIDDEN), jnp.float32),
                wt[HIDDEN + 3:]], axis=0)
    else:
        def arrange(wt):
            return jnp.concatenate([
                wt[:HIDDEN], wt[HIDDEN:HIDDEN + 3],
                jnp.zeros((5, HIDDEN), jnp.float32)], axis=0)
    w = jnp.concatenate([arrange(w0t), arrange(w1t)], axis=1)
    b = jnp.concatenate([w0b, w1b]).reshape(1, 256)
    return w.astype(jnp.bfloat16), b


def _pack_offset_w(p):
    oW, ob = p['vert_offset']
    ot = oW.T
    w = jnp.concatenate([ot[:HIDDEN], ot[HIDDEN:HIDDEN + 3],
                         jnp.zeros((5, 3), jnp.float32)], axis=0)
    w = jnp.concatenate([w, jnp.zeros((136, 5), jnp.float32)], axis=1)
    b = jnp.concatenate([ob, jnp.zeros((5,), jnp.float32)]).reshape(1, 8)
    return w.astype(jnp.bfloat16), b


def kernel(img_feats, verts_padded, edges_packed, params):
    f32 = jnp.float32
    img_t = jnp.transpose(img_feats, (0, 2, 3, 1)).reshape(N * H_IMG * W_IMG,
                                                           C_IMG)
    bws = [params['stages'][s]['bottleneck'][0].T for s in range(NUM_STAGES)]
    img_proj_all = _tc_imgproj(
        img_t, jnp.concatenate(bws, axis=1).astype(jnp.bfloat16))
    img_projs = [img_proj_all[:, s * 128:(s + 1) * 128] for s in
                 range(NUM_STAGES)]

    src = edges_packed[:, 0]
    dst = edges_packed[:, 1]
    gidx = jnp.concatenate([dst, src])
    sidx = jnp.concatenate([src, dst])
    gidx = jnp.concatenate([gidx, jnp.zeros((E2P - E2,), jnp.int32)])
    sidx = jnp.concatenate(
        [sidx, jnp.full((E2P - E2,), DUMMY_ROW, jnp.int32)])
    sidx2d = sidx.reshape(E2P // EB, EB)

    verts_flat = jnp.pad(verts_padded.reshape(VTOT, 3),
                         ((0, VP - VTOT), (0, 0)))
    pos8 = jnp.pad(verts_flat, ((0, 0), (0, 5)))

    outs = []
    vfeat = None
    for s in range(NUM_STAGES):
        sp = params['stages'][s]
        xs = pos8[:, 0].reshape(784, 128)
        ys = pos8[:, 1].reshape(784, 128)
        tap_idx, tap_w = _tc_prep(xs, ys)
        taps_flat = _sc_gather_taps(img_projs[s], tap_idx.reshape(4 * VP))
        w8 = jnp.pad(jnp.transpose(tap_w.reshape(4, VP)), ((0, 0), (0, 4)))
        bb = sp['bottleneck'][1].reshape(1, 128)
        va = _tc_va(taps_flat, w8, bb)

        if s == 0:
            w, b = _pack_gconv_w(sp['gconvs'][0], False)
            xw0, vw1 = _tc_gconv_first_s0(va, pos8, w, b)
        else:
            w, b = _pack_gconv_w(sp['gconvs'][0], True)
            xw0, vw1 = _tc_gconv_first(va, pos8, vfeat, w, b)
        agg = _sc_edge_agg(vw1, gidx, sidx2d)
        for gi in range(1, STAGE_DEPTH):
            w, b = _pack_gconv_w(sp['gconvs'][gi], False)
            xw0, vw1 = _tc_gconv_mid(xw0, agg, pos8, w, b)
            agg = _sc_edge_agg(vw1, gidx, sidx2d)

        ow, ob = _pack_offset_w(sp)
        verts8, vfeat = _tc_offset(xw0, agg, pos8, ow, ob)
        pos8 = verts8
        outs.append(verts8[:VTOT, :3].reshape(N, V, 3))

    return jnp.stack(outs, axis=0).astype(f32)


# speedup vs baseline: 3.6185x; 1.0004x over previous
import functools

import jax
import jax.numpy as jnp
from jax import lax
from jax.experimental import pallas as pl
from jax.experimental.pallas import tpu as pltpu
from jax.experimental.pallas import tpu_sc as plsc

N, V, E = 10, 10000, 300000
C_IMG, H_IMG, W_IMG = 256, 32, 32
HIDDEN = 128
NUM_STAGES = 3
STAGE_DEPTH = 3

VTOT = N * V
VP = 100352
RB = 512
NROW = VP // RB
E2 = 2 * E
SC_TILES = 16
EB = 128
EBATCH = 256
E_PER_TILE = 38912
E2P = SC_TILES * E_PER_TILE
NBATCH = E_PER_TILE // EBATCH
ACC_ROWS = VP
DUMMY_ROW = VTOT
VPT = VP // SC_TILES
ZCHUNK = 224
NCHUNKS = 8
TAPS_TOT = 4 * VP
TAP_ROWS_PER_TILE = TAPS_TOT // 32
TAP_BATCHES = TAP_ROWS_PER_TILE // EB


def _mm_imgproj_body(x_ref, w_ref, o_ref):
    o_ref[...] = jnp.dot(x_ref[...].astype(jnp.bfloat16), w_ref[...],
                         preferred_element_type=jnp.float32
                         ).astype(jnp.bfloat16)


def _tc_imgproj(img_t, w):
    return pl.pallas_call(
        _mm_imgproj_body,
        grid=(10240 // RB,),
        in_specs=[pl.BlockSpec((RB, C_IMG), lambda i: (i, 0)),
                  pl.BlockSpec((C_IMG, 384), lambda i: (0, 0))],
        out_specs=pl.BlockSpec((RB, 384), lambda i: (i, 0)),
        out_shape=jax.ShapeDtypeStruct((10240, 384), jnp.bfloat16),
    )(img_t, w)


def _prep_body(xs_ref, ys_ref, idx_ref, w_ref):
    gx = xs_ref[...]
    gy = ys_ref[...]
    x = (gx + 1.0) * ((W_IMG - 1) / 2.0)
    y = (1.0 - gy) * ((H_IMG - 1) / 2.0)
    x0 = jnp.floor(x)
    y0 = jnp.floor(y)
    wx1 = x - x0
    wx0 = 1.0 - wx1
    wy1 = y - y0
    wy0 = 1.0 - wy1
    vid = lax.broadcasted_iota(jnp.int32, (784, 128), 0) * 128 + \
        lax.broadcasted_iota(jnp.int32, (784, 128), 1)
    n = jnp.clip(vid // V, 0, N - 1)
    for t, (ix, iy, wt) in enumerate((
            (x0, y0, wx0 * wy0), (x0 + 1.0, y0, wx1 * wy0),
            (x0, y0 + 1.0, wx0 * wy1), (x0 + 1.0, y0 + 1.0, wx1 * wy1))):
        valid = ((ix >= 0.0) & (ix <= W_IMG - 1.0)
                 & (iy >= 0.0) & (iy <= H_IMG - 1.0))
        ixc = jnp.clip(ix, 0.0, W_IMG - 1.0).astype(jnp.int32)
        iyc = jnp.clip(iy, 0.0, H_IMG - 1.0).astype(jnp.int32)
        idx_ref[t] = n * (H_IMG * W_IMG) + iyc * W_IMG + ixc
        w_ref[t] = jnp.where(valid, wt, 0.0)


def _tc_prep(xs, ys):
    return pl.pallas_call(
        _prep_body,
        in_specs=[pl.BlockSpec((784, 128), lambda: (0, 0)),
                  pl.BlockSpec((784, 128), lambda: (0, 0))],
        out_specs=[pl.BlockSpec((4, 784, 128), lambda: (0, 0, 0)),
                   pl.BlockSpec((4, 784, 128), lambda: (0, 0, 0))],
        out_shape=[jax.ShapeDtypeStruct((4, 784, 128), jnp.int32),
                   jax.ShapeDtypeStruct((4, 784, 128), jnp.float32)],
    )(xs, ys)


def _va_body(t0, t1, t2, t3, w_ref, b_ref, o_ref):
    w = w_ref[...]
    acc = t0[...].astype(jnp.float32) * w[:, 0:1]
    acc += t1[...].astype(jnp.float32) * w[:, 1:2]
    acc += t2[...].astype(jnp.float32) * w[:, 2:3]
    acc += t3[...].astype(jnp.float32) * w[:, 3:4]
    o_ref[...] = jnp.maximum(acc + b_ref[...], 0.0).astype(jnp.bfloat16)


def _tc_va(taps_flat, w8, bb):
    specs = [pl.BlockSpec((RB, 128), functools.partial(
        lambda i, t: (t * NROW + i, 0), t=t)) for t in range(4)]
    return pl.pallas_call(
        _va_body,
        grid=(NROW,),
        in_specs=specs + [pl.BlockSpec((RB, 8), lambda i: (i, 0)),
                          pl.BlockSpec((1, 128), lambda i: (0, 0))],
        out_specs=pl.BlockSpec((RB, 128), lambda i: (i, 0)),
        out_shape=jax.ShapeDtypeStruct((VP, 128), jnp.bfloat16),
    )(taps_flat, taps_flat, taps_flat, taps_flat, w8, bb)


def _gconv_first_body(a_ref, p_ref, f_ref, w_ref, b_ref, o0_ref, o1_ref):
    xin = jnp.concatenate([a_ref[...], p_ref[...].astype(jnp.bfloat16),
                           f_ref[...]], axis=1)
    out = jnp.dot(xin, w_ref[...], preferred_element_type=jnp.float32)
    out += b_ref[...]
    o0_ref[...] = out[:, :HIDDEN].astype(jnp.bfloat16)
    o1_ref[...] = out[:, HIDDEN:].astype(jnp.bfloat16)


def _tc_gconv_first(va, pos8, vfeat, w, b):
    return pl.pallas_call(
        _gconv_first_body,
        grid=(NROW,),
        in_specs=[pl.BlockSpec((RB, 128), lambda i: (i, 0)),
                  pl.BlockSpec((RB, 8), lambda i: (i, 0)),
                  pl.BlockSpec((RB, 128), lambda i: (i, 0)),
                  pl.BlockSpec((264, 256), lambda i: (0, 0)),
                  pl.BlockSpec((1, 256), lambda i: (0, 0))],
        out_specs=[pl.BlockSpec((RB, 128), lambda i: (i, 0)),
                   pl.BlockSpec((RB, 128), lambda i: (i, 0))],
        out_shape=[jax.ShapeDtypeStruct((VP, 128), jnp.bfloat16),
                   jax.ShapeDtypeStruct((VP, 128), jnp.bfloat16)],
    )(va, pos8, vfeat, w, b)


def _gconv_first_s0_body(a_ref, p_ref, w_ref, b_ref, o0_ref, o1_ref):
    xin = jnp.concatenate([a_ref[...], p_ref[...].astype(jnp.bfloat16)],
                          axis=1)
    out = jnp.dot(xin, w_ref[...], preferred_element_type=jnp.float32)
    out += b_ref[...]
    o0_ref[...] = out[:, :HIDDEN].astype(jnp.bfloat16)
    o1_ref[...] = out[:, HIDDEN:].astype(jnp.bfloat16)


def _tc_gconv_first_s0(va, pos8, w, b):
    return pl.pallas_call(
        _gconv_first_s0_body,
        grid=(NROW,),
        in_specs=[pl.BlockSpec((RB, 128), lambda i: (i, 0)),
                  pl.BlockSpec((RB, 8), lambda i: (i, 0)),
                  pl.BlockSpec((136, 256), lambda i: (0, 0)),
                  pl.BlockSpec((1, 256), lambda i: (0, 0))],
        out_specs=[pl.BlockSpec((RB, 128), lambda i: (i, 0)),
                   pl.BlockSpec((RB, 128), lambda i: (i, 0))],
        out_shape=[jax.ShapeDtypeStruct((VP, 128), jnp.bfloat16),
                   jax.ShapeDtypeStruct((VP, 128), jnp.bfloat16)],
    )(va, pos8, w, b)


def _gconv_mid_body(x0_ref, g_ref, p_ref, w_ref, b_ref, o0_ref, o1_ref):
    nopos = jnp.maximum(x0_ref[...].astype(jnp.float32)
                        + g_ref[...].astype(jnp.float32), 0.0)
    xin = jnp.concatenate([nopos.astype(jnp.bfloat16),
                           p_ref[...].astype(jnp.bfloat16)], axis=1)
    out = jnp.dot(xin, w_ref[...], preferred_element_type=jnp.float32)
    out += b_ref[...]
    o0_ref[...] = out[:, :HIDDEN].astype(jnp.bfloat16)
    o1_ref[...] = out[:, HIDDEN:].astype(jnp.bfloat16)


def _tc_gconv_mid(xw0, agg, pos8, w, b):
    return pl.pallas_call(
        _gconv_mid_body,
        grid=(NROW,),
        in_specs=[pl.BlockSpec((RB, 128), lambda i: (i, 0)),
                  pl.BlockSpec((RB, 128), lambda i: (i, 0)),
                  pl.BlockSpec((RB, 8), lambda i: (i, 0)),
                  pl.BlockSpec((136, 256), lambda i: (0, 0)),
                  pl.BlockSpec((1, 256), lambda i: (0, 0))],
        out_specs=[pl.BlockSpec((RB, 128), lambda i: (i, 0)),
                   pl.BlockSpec((RB, 128), lambda i: (i, 0))],
        out_shape=[jax.ShapeDtypeStruct((VP, 128), jnp.bfloat16),
                   jax.ShapeDtypeStruct((VP, 128), jnp.bfloat16)],
    )(xw0, agg, pos8, w, b)


def _offset_body(x0_ref, g_ref, p_ref, w_ref, b_ref, v_ref, np_ref):
    nopos = jnp.maximum(x0_ref[...].astype(jnp.float32)
                        + g_ref[...].astype(jnp.float32), 0.0)
    np_ref[...] = nopos.astype(jnp.bfloat16)
    xin = jnp.concatenate([nopos.astype(jnp.bfloat16),
                           p_ref[...].astype(jnp.bfloat16)], axis=1)
    out = jnp.dot(xin, w_ref[...], preferred_element_type=jnp.float32)
    v_ref[...] = p_ref[...] + jnp.tanh(out + b_ref[...])


def _tc_offset(xw0, agg, pos8, w, b):
    return pl.pallas_call(
        _offset_body,
        grid=(NROW,),
        in_specs=[pl.BlockSpec((RB, 128), lambda i: (i, 0)),
                  pl.BlockSpec((RB, 128), lambda i: (i, 0)),
                  pl.BlockSpec((RB, 8), lambda i: (i, 0)),
                  pl.BlockSpec((136, 8), lambda i: (0, 0)),
                  pl.BlockSpec((1, 8), lambda i: (0, 0))],
        out_specs=[pl.BlockSpec((RB, 8), lambda i: (i, 0)),
                   pl.BlockSpec((RB, 128), lambda i: (i, 0))],
        out_shape=[jax.ShapeDtypeStruct((VP, 8), jnp.float32),
                   jax.ShapeDtypeStruct((VP, 128), jnp.bfloat16)],
    )(xw0, agg, pos8, w, b)


def _sc_mesh():
    return plsc.VectorSubcoreMesh(core_axis_name="c", subcore_axis_name="s",
                                  num_cores=2, num_subcores=16)


def _sc_taps_body(table_hbm, idx_hbm, out_hbm, idx_v, rows_v, sem_g, sem_w):
    cid = lax.axis_index("c")
    sid = lax.axis_index("s")
    wid = sid * 2 + cid
    base = wid * TAP_ROWS_PER_TILE
    pltpu.sync_copy(idx_hbm.at[pl.ds(base, TAP_ROWS_PER_TILE)], idx_v)
    MBR = 448
    NB = TAP_ROWS_PER_TILE // MBR
    NS = 3

    def fire_gather(j):
        pltpu.async_copy(table_hbm.at[idx_v.at[pl.ds(j * MBR, MBR)]],
                         rows_v.at[lax.rem(j, NS)], sem_g)

    def body(j, _):
        @pl.when((j >= 2) & (j - 2 < NB))
        def _():
            pltpu.make_async_copy(
                rows_v.at[lax.rem(j, NS)],
                out_hbm.at[pl.ds(base + (j - 2) * MBR, MBR)], sem_w).wait()

        @pl.when(j + 2 < NB)
        def _():
            fire_gather(j + 2)

        @pl.when(j < NB)
        def _():
            slot = lax.rem(j, NS)
            pltpu.make_async_copy(table_hbm.at[idx_v.at[pl.ds(j * MBR, MBR)]],
                                  rows_v.at[slot], sem_g).wait()
            pltpu.async_copy(rows_v.at[slot],
                             out_hbm.at[pl.ds(base + j * MBR, MBR)], sem_w)
        return 0

    for p in range(2):
        fire_gather(p)
    lax.fori_loop(0, NB + 2, body, 0, unroll=False)


def _sc_gather_taps(img_proj, tap_idx):
    kfn = pl.kernel(
        _sc_taps_body,
        out_type=jax.ShapeDtypeStruct((TAPS_TOT, 128), jnp.bfloat16),
        mesh=_sc_mesh(),
        scratch_types=[
            pltpu.VMEM((TAP_ROWS_PER_TILE,), jnp.int32),
            pltpu.VMEM((3, 448, 128), jnp.bfloat16),
            pltpu.SemaphoreType.DMA,
            pltpu.SemaphoreType.DMA,
        ],
        compiler_params=pltpu.CompilerParams(use_tc_tiling_on_sc=False),
    )
    return kfn(img_proj, tap_idx)


def _sc_agg_body(vw1r_hbm, gidx_hbm, sidx_hbm, out_hbm,
                 gbuf, ibuf, sbuf, rows_v, zbuf, acc,
                 sem_i, sem_g, sem_s, sem_z):
    cid = lax.axis_index("c")
    sid = lax.axis_index("s")
    ebase = sid * E_PER_TILE
    sbase = sid * (E_PER_TILE // EB)
    KPB = EBATCH // EB

    def zb(i, _):
        zbuf[i, :] = jnp.zeros((32,), jnp.bfloat16)
        return 0
    lax.fori_loop(0, ZCHUNK, zb, 0, unroll=False)

    def fire_idx(j):
        pltpu.async_copy(gidx_hbm.at[pl.ds(ebase + j * EBATCH, EBATCH)],
                         gbuf.at[lax.rem(j, 3)], sem_i)
        pltpu.async_copy(sidx_hbm.at[pl.ds(sbase + j * KPB, KPB)],
                         sbuf.at[lax.rem(j, 5)], sem_i)

    def drain_idx(j):
        pltpu.make_async_copy(gidx_hbm.at[pl.ds(0, EBATCH)],
                              gbuf.at[lax.rem(j, 3)], sem_i).wait()
        pltpu.make_async_copy(sidx_hbm.at[pl.ds(0, KPB)],
                              sbuf.at[lax.rem(j, 5)], sem_i).wait()

    def fire_gathers(j):
        for k in range(KPB):
            pltpu.async_copy(
                vw1r_hbm.at[ibuf.at[lax.rem(j, 3), k]],
                rows_v.at[lax.rem(j, 4), pl.ds(k * EB, EB)], sem_g)

    def drain_gathers(j):
        for k in range(KPB):
            pltpu.make_async_copy(
                vw1r_hbm.at[ibuf.at[lax.rem(j, 3), k]],
                rows_v.at[lax.rem(j, 4), pl.ds(k * EB, EB)], sem_g).wait()

    def fire_scatters(j):
        for k in range(KPB):
            pltpu.async_copy(
                rows_v.at[lax.rem(j, 4), pl.ds(k * EB, EB)],
                acc.at[sbuf.at[lax.rem(j, 5), k]], sem_s, add=True)

    def drain_scatters(j):
        for k in range(KPB):
            pltpu.make_async_copy(
                rows_v.at[lax.rem(j, 4), pl.ds(k * EB, EB)],
                acc.at[sbuf.at[lax.rem(j, 5), k]], sem_s).wait()

    def do_chunk(f):
        for z in range(VPT // ZCHUNK):
            pltpu.async_copy(
                zbuf, acc.at[pl.ds(sid * VPT + z * ZCHUNK, ZCHUNK)], sem_z)
        for z in range(VPT // ZCHUNK):
            pltpu.make_async_copy(
                zbuf, acc.at[pl.ds(sid * VPT + z * ZCHUNK, ZCHUNK)],
                sem_z).wait()
        plsc.subcore_barrier()

        def batch(j, _):
            @pl.when((j >= 3) & (j - 3 < NBATCH))
            def _():
                drain_scatters(j - 3)

            @pl.when(j + 2 < NBATCH)
            def _():
                fire_idx(j + 2)

            @pl.when(j < NBATCH)
            def _():
                drain_idx(j)
                for k in range(KPB):
                    for i in range(EB // 16):
                        g = gbuf[lax.rem(j, 3), pl.ds(k * EB + i * 16, 16)]
                        ibuf[lax.rem(j, 3), k, pl.ds(i * 16, 16)] = \
                            g * 4 + f
                fire_gathers(j)

            @pl.when((j >= 2) & (j - 2 < NBATCH))
            def _():
                drain_gathers(j - 2)
                fire_scatters(j - 2)
            return 0

        fire_idx(0)
        fire_idx(1)
        lax.fori_loop(0, NBATCH + 3, batch, 0, unroll=False)
        plsc.subcore_barrier()
        pltpu.sync_copy(
            acc.at[pl.ds(sid * VPT, VPT)],
            out_hbm.at[pl.ds(sid * VPT, VPT), pl.ds(f * 32, 32)])
        plsc.subcore_barrier()

    for fi in range(2):
        do_chunk(cid * 2 + fi)


def _sc_edge_agg(vw1, gidx, sidx2d):
    kfn = pl.kernel(
        _sc_agg_body,
        out_type=jax.ShapeDtypeStruct((VP, 128), jnp.bfloat16),
        mesh=_sc_mesh(),
        scratch_types=[
            pltpu.VMEM((3, EBATCH), jnp.int32),
            pltpu.VMEM((3, EBATCH // EB, EB), jnp.int32),
            pltpu.VMEM((5, EBATCH // EB, EB), jnp.int32),
            pltpu.VMEM((4, EBATCH, 32), jnp.bfloat16),
            pltpu.VMEM((ZCHUNK, 32), jnp.bfloat16),
            pltpu.VMEM_SHARED((ACC_ROWS, 32), jnp.bfloat16),
            pltpu.SemaphoreType.DMA,
            pltpu.SemaphoreType.DMA,
            pltpu.SemaphoreType.DMA,
            pltpu.SemaphoreType.DMA,
        ],
        compiler_params=pltpu.CompilerParams(use_tc_tiling_on_sc=False),
    )
    return kfn(vw1.reshape(VP * 4, 32), gidx, sidx2d)


def _pack_gconv_w(p, first_with_feats):
    w0W, w0b = p['w0']
    w1W, w1b = p['w1']
    w0t, w1t = w0W.T, w1W.T
    if first_with_feats:
        def arrange(wt):
            return jnp.concatenate([
                wt[:HIDDEN], wt[HIDDEN:HIDDEN + 3],
                jnp.zeros((5, HIDDEN), jnp.float32),
                wt[HIDDEN + 3:]], axis=0)
    else:
        def arrange(wt):
            return jnp.concatenate([
                wt[:HIDDEN], wt[HIDDEN:HIDDEN + 3],
                jnp.zeros((5, HIDDEN), jnp.float32)], axis=0)
    w = jnp.concatenate([arrange(w0t), arrange(w1t)], axis=1)
    b = jnp.concatenate([w0b, w1b]).reshape(1, 256)
    return w.astype(jnp.bfloat16), b


def _pack_offset_w(p):
    oW, ob = p['vert_offset']
    ot = oW.T
    w = jnp.concatenate([ot[:HIDDEN], ot[HIDDEN:HIDDEN + 3],
                         jnp.zeros((5, 3), jnp.float32)], axis=0)
    w = jnp.concatenate([w, jnp.zeros((136, 5), jnp.float32)], axis=1)
    b = jnp.concatenate([ob, jnp.zeros((5,), jnp.float32)]).reshape(1, 8)
    return w.astype(jnp.bfloat16), b


def kernel(img_feats, verts_padded, edges_packed, params):
    f32 = jnp.float32
    img_t = jnp.transpose(img_feats, (0, 2, 3, 1)).reshape(N * H_IMG * W_IMG,
                                                           C_IMG)
    bws = [params['stages'][s]['bottleneck'][0].T for s in range(NUM_STAGES)]
    img_proj_all = _tc_imgproj(
        img_t, jnp.concatenate(bws, axis=1).astype(jnp.bfloat16))
    img_projs = [img_proj_all[:, s * 128:(s + 1) * 128] for s in
                 range(NUM_STAGES)]

    src = edges_packed[:, 0]
    dst = edges_packed[:, 1]
    gidx = jnp.concatenate([dst, src])
    sidx = jnp.concatenate([src, dst])
    gidx = jnp.concatenate([gidx, jnp.zeros((E2P - E2,), jnp.int32)])
    sidx = jnp.concatenate(
        [sidx, jnp.full((E2P - E2,), DUMMY_ROW, jnp.int32)])
    sidx2d = sidx.reshape(E2P // EB, EB)

    verts_flat = jnp.pad(verts_padded.reshape(VTOT, 3),
                         ((0, VP - VTOT), (0, 0)))
    pos8 = jnp.pad(verts_flat, ((0, 0), (0, 5)))

    outs = []
    vfeat = None
    for s in range(NUM_STAGES):
        sp = params['stages'][s]
        xs = pos8[:, 0].reshape(784, 128)
        ys = pos8[:, 1].reshape(784, 128)
        tap_idx, tap_w = _tc_prep(xs, ys)
        taps_flat = _sc_gather_taps(img_projs[s], tap_idx.reshape(4 * VP))
        w8 = jnp.pad(jnp.transpose(tap_w.reshape(4, VP)), ((0, 0), (0, 4)))
        bb = sp['bottleneck'][1].reshape(1, 128)
        va = _tc_va(taps_flat, w8, bb)

        if s == 0:
            w, b = _pack_gconv_w(sp['gconvs'][0], False)
            xw0, vw1 = _tc_gconv_first_s0(va, pos8, w, b)
        else:
            w, b = _pack_gconv_w(sp['gconvs'][0], True)
            xw0, vw1 = _tc_gconv_first(va, pos8, vfeat, w, b)
        agg = _sc_edge_agg(vw1, gidx, sidx2d)
        for gi in range(1, STAGE_DEPTH):
            w, b = _pack_gconv_w(sp['gconvs'][gi], False)
            xw0, vw1 = _tc_gconv_mid(xw0, agg, pos8, w, b)
            agg = _sc_edge_agg(vw1, gidx, sidx2d)

        ow, ob = _pack_offset_w(sp)
        verts8, vfeat = _tc_offset(xw0, agg, pos8, ow, ob)
        pos8 = verts8
        outs.append(verts8[:VTOT, :3].reshape(N, V, 3))

    return jnp.stack(outs, axis=0).astype(f32)


# speedup vs baseline: 3.7288x; 1.0305x over previous
import functools

import jax
import jax.numpy as jnp
from jax import lax
from jax.experimental import pallas as pl
from jax.experimental.pallas import tpu as pltpu
from jax.experimental.pallas import tpu_sc as plsc

N, V, E = 10, 10000, 300000
C_IMG, H_IMG, W_IMG = 256, 32, 32
HIDDEN = 128
NUM_STAGES = 3
STAGE_DEPTH = 3

VTOT = N * V
VP = 100352
RB = 512
NROW = VP // RB
E2 = 2 * E
SC_TILES = 16
EB = 128
EBATCH = 256
E_PER_TILE = 38912
E2P = SC_TILES * E_PER_TILE
NBATCH = E_PER_TILE // EBATCH
ACC_ROWS = VP
DUMMY_ROW = VTOT
VPT = VP // SC_TILES
ZCHUNK = 224
NCHUNKS = 8
TAPS_TOT = 4 * VP
TAP_ROWS_PER_TILE = TAPS_TOT // 32
TAP_BATCHES = TAP_ROWS_PER_TILE // EB


def _mm_imgproj_body(x_ref, w_ref, o_ref):
    o_ref[...] = jnp.dot(x_ref[...].astype(jnp.bfloat16), w_ref[...],
                         preferred_element_type=jnp.float32
                         ).astype(jnp.bfloat16)


def _tc_imgproj(img_t, w):
    return pl.pallas_call(
        _mm_imgproj_body,
        grid=(10240 // RB,),
        in_specs=[pl.BlockSpec((RB, C_IMG), lambda i: (i, 0)),
                  pl.BlockSpec((C_IMG, 384), lambda i: (0, 0))],
        out_specs=pl.BlockSpec((RB, 384), lambda i: (i, 0)),
        out_shape=jax.ShapeDtypeStruct((10240, 384), jnp.bfloat16),
    )(img_t, w)


def _prep_body(xs_ref, ys_ref, idx_ref, w_ref):
    gx = xs_ref[...]
    gy = ys_ref[...]
    x = (gx + 1.0) * ((W_IMG - 1) / 2.0)
    y = (1.0 - gy) * ((H_IMG - 1) / 2.0)
    x0 = jnp.floor(x)
    y0 = jnp.floor(y)
    wx1 = x - x0
    wx0 = 1.0 - wx1
    wy1 = y - y0
    wy0 = 1.0 - wy1
    vid = lax.broadcasted_iota(jnp.int32, (784, 128), 0) * 128 + \
        lax.broadcasted_iota(jnp.int32, (784, 128), 1)
    n = jnp.clip(vid // V, 0, N - 1)
    for t, (ix, iy, wt) in enumerate((
            (x0, y0, wx0 * wy0), (x0 + 1.0, y0, wx1 * wy0),
            (x0, y0 + 1.0, wx0 * wy1), (x0 + 1.0, y0 + 1.0, wx1 * wy1))):
        valid = ((ix >= 0.0) & (ix <= W_IMG - 1.0)
                 & (iy >= 0.0) & (iy <= H_IMG - 1.0))
        ixc = jnp.clip(ix, 0.0, W_IMG - 1.0).astype(jnp.int32)
        iyc = jnp.clip(iy, 0.0, H_IMG - 1.0).astype(jnp.int32)
        idx_ref[t] = n * (H_IMG * W_IMG) + iyc * W_IMG + ixc
        w_ref[t] = jnp.where(valid, wt, 0.0)


def _tc_prep(xs, ys):
    return pl.pallas_call(
        _prep_body,
        in_specs=[pl.BlockSpec((784, 128), lambda: (0, 0)),
                  pl.BlockSpec((784, 128), lambda: (0, 0))],
        out_specs=[pl.BlockSpec((4, 784, 128), lambda: (0, 0, 0)),
                   pl.BlockSpec((4, 784, 128), lambda: (0, 0, 0))],
        out_shape=[jax.ShapeDtypeStruct((4, 784, 128), jnp.int32),
                   jax.ShapeDtypeStruct((4, 784, 128), jnp.float32)],
    )(xs, ys)


def _va_body(t0, t1, t2, t3, w_ref, b_ref, o_ref):
    w = w_ref[...]
    acc = t0[...].astype(jnp.float32) * w[:, 0:1]
    acc += t1[...].astype(jnp.float32) * w[:, 1:2]
    acc += t2[...].astype(jnp.float32) * w[:, 2:3]
    acc += t3[...].astype(jnp.float32) * w[:, 3:4]
    o_ref[...] = jnp.maximum(acc + b_ref[...], 0.0).astype(jnp.bfloat16)


def _tc_va(taps_flat, w8, bb):
    specs = [pl.BlockSpec((RB, 128), functools.partial(
        lambda i, t: (t * NROW + i, 0), t=t)) for t in range(4)]
    return pl.pallas_call(
        _va_body,
        grid=(NROW,),
        in_specs=specs + [pl.BlockSpec((RB, 8), lambda i: (i, 0)),
                          pl.BlockSpec((1, 128), lambda i: (0, 0))],
        out_specs=pl.BlockSpec((RB, 128), lambda i: (i, 0)),
        out_shape=jax.ShapeDtypeStruct((VP, 128), jnp.bfloat16),
    )(taps_flat, taps_flat, taps_flat, taps_flat, w8, bb)


def _gconv_first_body(a_ref, p_ref, f_ref, w_ref, b_ref, o0_ref, o1_ref):
    xin = jnp.concatenate([a_ref[...], p_ref[...].astype(jnp.bfloat16),
                           f_ref[...]], axis=1)
    out = jnp.dot(xin, w_ref[...], preferred_element_type=jnp.float32)
    out += b_ref[...]
    o0_ref[...] = out[:, :HIDDEN].astype(jnp.bfloat16)
    o1_ref[...] = out[:, HIDDEN:].astype(jnp.bfloat16)


def _tc_gconv_first(va, pos8, vfeat, w, b):
    return pl.pallas_call(
        _gconv_first_body,
        grid=(NROW,),
        in_specs=[pl.BlockSpec((RB, 128), lambda i: (i, 0)),
                  pl.BlockSpec((RB, 8), lambda i: (i, 0)),
                  pl.BlockSpec((RB, 128), lambda i: (i, 0)),
                  pl.BlockSpec((264, 256), lambda i: (0, 0)),
                  pl.BlockSpec((1, 256), lambda i: (0, 0))],
        out_specs=[pl.BlockSpec((RB, 128), lambda i: (i, 0)),
                   pl.BlockSpec((RB, 128), lambda i: (i, 0))],
        out_shape=[jax.ShapeDtypeStruct((VP, 128), jnp.bfloat16),
                   jax.ShapeDtypeStruct((VP, 128), jnp.bfloat16)],
    )(va, pos8, vfeat, w, b)


def _gconv_first_s0_body(a_ref, p_ref, w_ref, b_ref, o0_ref, o1_ref):
    xin = jnp.concatenate([a_ref[...], p_ref[...].astype(jnp.bfloat16)],
                          axis=1)
    out = jnp.dot(xin, w_ref[...], preferred_element_type=jnp.float32)
    out += b_ref[...]
    o0_ref[...] = out[:, :HIDDEN].astype(jnp.bfloat16)
    o1_ref[...] = out[:, HIDDEN:].astype(jnp.bfloat16)


def _tc_gconv_first_s0(va, pos8, w, b):
    return pl.pallas_call(
        _gconv_first_s0_body,
        grid=(NROW,),
        in_specs=[pl.BlockSpec((RB, 128), lambda i: (i, 0)),
                  pl.BlockSpec((RB, 8), lambda i: (i, 0)),
                  pl.BlockSpec((136, 256), lambda i: (0, 0)),
                  pl.BlockSpec((1, 256), lambda i: (0, 0))],
        out_specs=[pl.BlockSpec((RB, 128), lambda i: (i, 0)),
                   pl.BlockSpec((RB, 128), lambda i: (i, 0))],
        out_shape=[jax.ShapeDtypeStruct((VP, 128), jnp.bfloat16),
                   jax.ShapeDtypeStruct((VP, 128), jnp.bfloat16)],
    )(va, pos8, w, b)


def _va_block(t0, t1, t2, t3, w_ref, b_ref):
    w = w_ref[...]
    acc = t0[...].astype(jnp.float32) * w[:, 0:1]
    acc += t1[...].astype(jnp.float32) * w[:, 1:2]
    acc += t2[...].astype(jnp.float32) * w[:, 2:3]
    acc += t3[...].astype(jnp.float32) * w[:, 3:4]
    return jnp.maximum(acc + b_ref[...], 0.0).astype(jnp.bfloat16)


def _gconv0_s0_body(t0, t1, t2, t3, tw_ref, bb_ref, p_ref, w_ref, b_ref,
                    o0_ref, o1_ref):
    va = _va_block(t0, t1, t2, t3, tw_ref, bb_ref)
    xin = jnp.concatenate([va, p_ref[...].astype(jnp.bfloat16)], axis=1)
    out = jnp.dot(xin, w_ref[...], preferred_element_type=jnp.float32)
    out += b_ref[...]
    o0_ref[...] = out[:, :HIDDEN].astype(jnp.bfloat16)
    o1_ref[...] = out[:, HIDDEN:].astype(jnp.bfloat16)


def _gconv0_body(t0, t1, t2, t3, tw_ref, bb_ref, p_ref, f_ref, w_ref, b_ref,
                 o0_ref, o1_ref):
    va = _va_block(t0, t1, t2, t3, tw_ref, bb_ref)
    xin = jnp.concatenate([va, p_ref[...].astype(jnp.bfloat16), f_ref[...]],
                          axis=1)
    out = jnp.dot(xin, w_ref[...], preferred_element_type=jnp.float32)
    out += b_ref[...]
    o0_ref[...] = out[:, :HIDDEN].astype(jnp.bfloat16)
    o1_ref[...] = out[:, HIDDEN:].astype(jnp.bfloat16)


def _tc_gconv0(taps_flat, w8, bb, pos8, vfeat, w, b):
    tspecs = [pl.BlockSpec((RB, 128), functools.partial(
        lambda i, t: (t * NROW + i, 0), t=t)) for t in range(4)]
    common = [pl.BlockSpec((RB, 8), lambda i: (i, 0)),
              pl.BlockSpec((1, 128), lambda i: (0, 0)),
              pl.BlockSpec((RB, 8), lambda i: (i, 0))]
    if vfeat is None:
        body = _gconv0_s0_body
        ins = common + [pl.BlockSpec((136, 256), lambda i: (0, 0)),
                        pl.BlockSpec((1, 256), lambda i: (0, 0))]
        args = (taps_flat,) * 4 + (w8, bb, pos8, w, b)
    else:
        body = _gconv0_body
        ins = common[:2] + [pl.BlockSpec((RB, 8), lambda i: (i, 0)),
                            pl.BlockSpec((RB, 128), lambda i: (i, 0)),
                            pl.BlockSpec((264, 256), lambda i: (0, 0)),
                            pl.BlockSpec((1, 256), lambda i: (0, 0))]
        args = (taps_flat,) * 4 + (w8, bb, pos8, vfeat, w, b)
    return pl.pallas_call(
        body,
        grid=(NROW,),
        in_specs=tspecs + ins,
        out_specs=[pl.BlockSpec((RB, 128), lambda i: (i, 0)),
                   pl.BlockSpec((RB, 128), lambda i: (i, 0))],
        out_shape=[jax.ShapeDtypeStruct((VP, 128), jnp.bfloat16),
                   jax.ShapeDtypeStruct((VP, 128), jnp.bfloat16)],
    )(*args)


def _gconv_mid_body(x0_ref, g_ref, p_ref, w_ref, b_ref, o0_ref, o1_ref):
    nopos = jnp.maximum(x0_ref[...].astype(jnp.float32)
                        + g_ref[...].astype(jnp.float32), 0.0)
    xin = jnp.concatenate([nopos.astype(jnp.bfloat16),
                           p_ref[...].astype(jnp.bfloat16)], axis=1)
    out = jnp.dot(xin, w_ref[...], preferred_element_type=jnp.float32)
    out += b_ref[...]
    o0_ref[...] = out[:, :HIDDEN].astype(jnp.bfloat16)
    o1_ref[...] = out[:, HIDDEN:].astype(jnp.bfloat16)


def _tc_gconv_mid(xw0, agg, pos8, w, b):
    return pl.pallas_call(
        _gconv_mid_body,
        grid=(NROW,),
        in_specs=[pl.BlockSpec((RB, 128), lambda i: (i, 0)),
                  pl.BlockSpec((RB, 128), lambda i: (i, 0)),
                  pl.BlockSpec((RB, 8), lambda i: (i, 0)),
                  pl.BlockSpec((136, 256), lambda i: (0, 0)),
                  pl.BlockSpec((1, 256), lambda i: (0, 0))],
        out_specs=[pl.BlockSpec((RB, 128), lambda i: (i, 0)),
                   pl.BlockSpec((RB, 128), lambda i: (i, 0))],
        out_shape=[jax.ShapeDtypeStruct((VP, 128), jnp.bfloat16),
                   jax.ShapeDtypeStruct((VP, 128), jnp.bfloat16)],
    )(xw0, agg, pos8, w, b)


def _offset_body(x0_ref, g_ref, p_ref, w_ref, b_ref, v_ref, np_ref):
    nopos = jnp.maximum(x0_ref[...].astype(jnp.float32)
                        + g_ref[...].astype(jnp.float32), 0.0)
    np_ref[...] = nopos.astype(jnp.bfloat16)
    xin = jnp.concatenate([nopos.astype(jnp.bfloat16),
                           p_ref[...].astype(jnp.bfloat16)], axis=1)
    out = jnp.dot(xin, w_ref[...], preferred_element_type=jnp.float32)
    v_ref[...] = p_ref[...] + jnp.tanh(out + b_ref[...])


def _tc_offset(xw0, agg, pos8, w, b):
    return pl.pallas_call(
        _offset_body,
        grid=(NROW,),
        in_specs=[pl.BlockSpec((RB, 128), lambda i: (i, 0)),
                  pl.BlockSpec((RB, 128), lambda i: (i, 0)),
                  pl.BlockSpec((RB, 8), lambda i: (i, 0)),
                  pl.BlockSpec((136, 8), lambda i: (0, 0)),
                  pl.BlockSpec((1, 8), lambda i: (0, 0))],
        out_specs=[pl.BlockSpec((RB, 8), lambda i: (i, 0)),
                   pl.BlockSpec((RB, 128), lambda i: (i, 0))],
        out_shape=[jax.ShapeDtypeStruct((VP, 8), jnp.float32),
                   jax.ShapeDtypeStruct((VP, 128), jnp.bfloat16)],
    )(xw0, agg, pos8, w, b)


def _sc_mesh():
    return plsc.VectorSubcoreMesh(core_axis_name="c", subcore_axis_name="s",
                                  num_cores=2, num_subcores=16)


def _sc_taps_body(table_hbm, idx_hbm, out_hbm, idx_v, rows_v, sem_g, sem_w):
    cid = lax.axis_index("c")
    sid = lax.axis_index("s")
    wid = sid * 2 + cid
    base = wid * TAP_ROWS_PER_TILE
    pltpu.sync_copy(idx_hbm.at[pl.ds(base, TAP_ROWS_PER_TILE)], idx_v)
    MBR = 448
    NB = TAP_ROWS_PER_TILE // MBR
    NS = 3

    def fire_gather(j):
        pltpu.async_copy(table_hbm.at[idx_v.at[pl.ds(j * MBR, MBR)]],
                         rows_v.at[lax.rem(j, NS)], sem_g)

    def body(j, _):
        @pl.when((j >= 2) & (j - 2 < NB))
        def _():
            pltpu.make_async_copy(
                rows_v.at[lax.rem(j, NS)],
                out_hbm.at[pl.ds(base + (j - 2) * MBR, MBR)], sem_w).wait()

        @pl.when(j + 2 < NB)
        def _():
            fire_gather(j + 2)

        @pl.when(j < NB)
        def _():
            slot = lax.rem(j, NS)
            pltpu.make_async_copy(table_hbm.at[idx_v.at[pl.ds(j * MBR, MBR)]],
                                  rows_v.at[slot], sem_g).wait()
            pltpu.async_copy(rows_v.at[slot],
                             out_hbm.at[pl.ds(base + j * MBR, MBR)], sem_w)
        return 0

    for p in range(2):
        fire_gather(p)
    lax.fori_loop(0, NB + 2, body, 0, unroll=False)


def _sc_gather_taps(img_proj, tap_idx):
    kfn = pl.kernel(
        _sc_taps_body,
        out_type=jax.ShapeDtypeStruct((TAPS_TOT, 128), jnp.bfloat16),
        mesh=_sc_mesh(),
        scratch_types=[
            pltpu.VMEM((TAP_ROWS_PER_TILE,), jnp.int32),
            pltpu.VMEM((3, 448, 128), jnp.bfloat16),
            pltpu.SemaphoreType.DMA,
            pltpu.SemaphoreType.DMA,
        ],
        compiler_params=pltpu.CompilerParams(use_tc_tiling_on_sc=False),
    )
    return kfn(img_proj, tap_idx)


def _sc_agg_body(vw1r_hbm, gidx_hbm, sidx_hbm, out_hbm,
                 gbuf, ibuf, sbuf, rows_v, zbuf, acc,
                 sem_i, sem_g, sem_s, sem_z):
    cid = lax.axis_index("c")
    sid = lax.axis_index("s")
    ebase = sid * E_PER_TILE
    sbase = sid * (E_PER_TILE // EB)
    KPB = EBATCH // EB

    def zb(i, _):
        zbuf[i, :] = jnp.zeros((32,), jnp.bfloat16)
        return 0
    lax.fori_loop(0, ZCHUNK, zb, 0, unroll=False)

    def fire_idx(j):
        pltpu.async_copy(gidx_hbm.at[pl.ds(ebase + j * EBATCH, EBATCH)],
                         gbuf.at[lax.rem(j, 3)], sem_i)
        pltpu.async_copy(sidx_hbm.at[pl.ds(sbase + j * KPB, KPB)],
                         sbuf.at[lax.rem(j, 5)], sem_i)

    def drain_idx(j):
        pltpu.make_async_copy(gidx_hbm.at[pl.ds(0, EBATCH)],
                              gbuf.at[lax.rem(j, 3)], sem_i).wait()
        pltpu.make_async_copy(sidx_hbm.at[pl.ds(0, KPB)],
                              sbuf.at[lax.rem(j, 5)], sem_i).wait()

    def fire_gathers(j):
        for k in range(KPB):
            pltpu.async_copy(
                vw1r_hbm.at[ibuf.at[lax.rem(j, 3), k]],
                rows_v.at[lax.rem(j, 4), pl.ds(k * EB, EB)], sem_g)

    def drain_gathers(j):
        for k in range(KPB):
            pltpu.make_async_copy(
                vw1r_hbm.at[ibuf.at[lax.rem(j, 3), k]],
                rows_v.at[lax.rem(j, 4), pl.ds(k * EB, EB)], sem_g).wait()

    def fire_scatters(j):
        for k in range(KPB):
            pltpu.async_copy(
                rows_v.at[lax.rem(j, 4), pl.ds(k * EB, EB)],
                acc.at[sbuf.at[lax.rem(j, 5), k]], sem_s, add=True)

    def drain_scatters(j):
        for k in range(KPB):
            pltpu.make_async_copy(
                rows_v.at[lax.rem(j, 4), pl.ds(k * EB, EB)],
                acc.at[sbuf.at[lax.rem(j, 5), k]], sem_s).wait()

    def do_chunk(f):
        for z in range(VPT // ZCHUNK):
            pltpu.async_copy(
                zbuf, acc.at[pl.ds(sid * VPT + z * ZCHUNK, ZCHUNK)], sem_z)
        for z in range(VPT // ZCHUNK):
            pltpu.make_async_copy(
                zbuf, acc.at[pl.ds(sid * VPT + z * ZCHUNK, ZCHUNK)],
                sem_z).wait()
        plsc.subcore_barrier()

        def batch(j, _):
            @pl.when((j >= 3) & (j - 3 < NBATCH))
            def _():
                drain_scatters(j - 3)

            @pl.when(j + 2 < NBATCH)
            def _():
                fire_idx(j + 2)

            @pl.when(j < NBATCH)
            def _():
                drain_idx(j)
                for k in range(KPB):
                    for i in range(EB // 16):
                        g = gbuf[lax.rem(j, 3), pl.ds(k * EB + i * 16, 16)]
                        ibuf[lax.rem(j, 3), k, pl.ds(i * 16, 16)] = \
                            g * 4 + f
                fire_gathers(j)

            @pl.when((j >= 2) & (j - 2 < NBATCH))
            def _():
                drain_gathers(j - 2)
                fire_scatters(j - 2)
            return 0

        fire_idx(0)
        fire_idx(1)
        lax.fori_loop(0, NBATCH + 3, batch, 0, unroll=False)
        plsc.subcore_barrier()
        pltpu.sync_copy(
            acc.at[pl.ds(sid * VPT, VPT)],
            out_hbm.at[pl.ds(sid * VPT, VPT), pl.ds(f * 32, 32)])
        plsc.subcore_barrier()

    for fi in range(2):
        do_chunk(cid * 2 + fi)


def _sc_edge_agg(vw1, gidx, sidx2d):
    kfn = pl.kernel(
        _sc_agg_body,
        out_type=jax.ShapeDtypeStruct((VP, 128), jnp.bfloat16),
        mesh=_sc_mesh(),
        scratch_types=[
            pltpu.VMEM((3, EBATCH), jnp.int32),
            pltpu.VMEM((3, EBATCH // EB, EB), jnp.int32),
            pltpu.VMEM((5, EBATCH // EB, EB), jnp.int32),
            pltpu.VMEM((4, EBATCH, 32), jnp.bfloat16),
            pltpu.VMEM((ZCHUNK, 32), jnp.bfloat16),
            pltpu.VMEM_SHARED((ACC_ROWS, 32), jnp.bfloat16),
            pltpu.SemaphoreType.DMA,
            pltpu.SemaphoreType.DMA,
            pltpu.SemaphoreType.DMA,
            pltpu.SemaphoreType.DMA,
        ],
        compiler_params=pltpu.CompilerParams(use_tc_tiling_on_sc=False),
    )
    return kfn(vw1.reshape(VP * 4, 32), gidx, sidx2d)


def _pack_gconv_w(p, first_with_feats):
    w0W, w0b = p['w0']
    w1W, w1b = p['w1']
    w0t, w1t = w0W.T, w1W.T
    if first_with_feats:
        def arrange(wt):
            return jnp.concatenate([
                wt[:HIDDEN], wt[HIDDEN:HIDDEN + 3],
                jnp.zeros((5, HIDDEN), jnp.float32),
                wt[HIDDEN + 3:]], axis=0)
    else:
        def arrange(wt):
            return jnp.concatenate([
                wt[:HIDDEN], wt[HIDDEN:HIDDEN + 3],
                jnp.zeros((5, HIDDEN), jnp.float32)], axis=0)
    w = jnp.concatenate([arrange(w0t), arrange(w1t)], axis=1)
    b = jnp.concatenate([w0b, w1b]).reshape(1, 256)
    return w.astype(jnp.bfloat16), b


def _pack_offset_w(p):
    oW, ob = p['vert_offset']
    ot = oW.T
    w = jnp.concatenate([ot[:HIDDEN], ot[HIDDEN:HIDDEN + 3],
                         jnp.zeros((5, 3), jnp.float32)], axis=0)
    w = jnp.concatenate([w, jnp.zeros((136, 5), jnp.float32)], axis=1)
    b = jnp.concatenate([ob, jnp.zeros((5,), jnp.float32)]).reshape(1, 8)
    return w.astype(jnp.bfloat16), b


def kernel(img_feats, verts_padded, edges_packed, params):
    f32 = jnp.float32
    img_t = jnp.transpose(img_feats, (0, 2, 3, 1)).reshape(N * H_IMG * W_IMG,
                                                           C_IMG)
    bws = [params['stages'][s]['bottleneck'][0].T for s in range(NUM_STAGES)]
    img_proj_all = _tc_imgproj(
        img_t, jnp.concatenate(bws, axis=1).astype(jnp.bfloat16))
    img_projs = [img_proj_all[:, s * 128:(s + 1) * 128] for s in
                 range(NUM_STAGES)]

    src = edges_packed[:, 0]
    dst = edges_packed[:, 1]
    gidx = jnp.concatenate([dst, src])
    sidx = jnp.concatenate([src, dst])
    gidx = jnp.concatenate([gidx, jnp.zeros((E2P - E2,), jnp.int32)])
    sidx = jnp.concatenate(
        [sidx, jnp.full((E2P - E2,), DUMMY_ROW, jnp.int32)])
    sidx2d = sidx.reshape(E2P // EB, EB)

    verts_flat = jnp.pad(verts_padded.reshape(VTOT, 3),
                         ((0, VP - VTOT), (0, 0)))
    pos8 = jnp.pad(verts_flat, ((0, 0), (0, 5)))

    outs = []
    vfeat = None
    for s in range(NUM_STAGES):
        sp = params['stages'][s]
        xs = pos8[:, 0].reshape(784, 128)
        ys = pos8[:, 1].reshape(784, 128)
        tap_idx, tap_w = _tc_prep(xs, ys)
        taps_flat = _sc_gather_taps(img_projs[s], tap_idx.reshape(4 * VP))
        w8 = jnp.pad(jnp.transpose(tap_w.reshape(4, VP)), ((0, 0), (0, 4)))
        bb = sp['bottleneck'][1].reshape(1, 128)

        w, b = _pack_gconv_w(sp['gconvs'][0], s > 0)
        xw0, vw1 = _tc_gconv0(taps_flat, w8, bb, pos8,
                              vfeat if s > 0 else None, w, b)
        agg = _sc_edge_agg(vw1, gidx, sidx2d)
        for gi in range(1, STAGE_DEPTH):
            w, b = _pack_gconv_w(sp['gconvs'][gi], False)
            xw0, vw1 = _tc_gconv_mid(xw0, agg, pos8, w, b)
            agg = _sc_edge_agg(vw1, gidx, sidx2d)

        ow, ob = _pack_offset_w(sp)
        verts8, vfeat = _tc_offset(xw0, agg, pos8, ow, ob)
        pos8 = verts8
        outs.append(verts8[:VTOT, :3].reshape(N, V, 3))

    return jnp.stack(outs, axis=0).astype(f32)


# speedup vs baseline: 3.7752x; 1.0124x over previous
import functools

import jax
import jax.numpy as jnp
from jax import lax
from jax.experimental import pallas as pl
from jax.experimental.pallas import tpu as pltpu
from jax.experimental.pallas import tpu_sc as plsc

N, V, E = 10, 10000, 300000
C_IMG, H_IMG, W_IMG = 256, 32, 32
HIDDEN = 128
NUM_STAGES = 3
STAGE_DEPTH = 3

VTOT = N * V
VP = 100352
RB = 512
NROW = VP // RB
E2 = 2 * E
SC_TILES = 16
EB = 128
EBATCH = 256
E_PER_TILE = 38912
E2P = SC_TILES * E_PER_TILE
NBATCH = E_PER_TILE // EBATCH
ACC_ROWS = VP
DUMMY_ROW = VTOT
VPT = VP // SC_TILES
ZCHUNK = 224
NCHUNKS = 8
TAPS_TOT = 4 * VP
TAP_ROWS_PER_TILE = TAPS_TOT // 32
TAP_BATCHES = TAP_ROWS_PER_TILE // EB


def _mm_imgproj_body(x_ref, w_ref, o_ref):
    o_ref[...] = jnp.dot(x_ref[...].astype(jnp.bfloat16), w_ref[...],
                         preferred_element_type=jnp.float32
                         ).astype(jnp.bfloat16)


def _tc_imgproj(img_t, w):
    return pl.pallas_call(
        _mm_imgproj_body,
        grid=(10240 // RB,),
        in_specs=[pl.BlockSpec((RB, C_IMG), lambda i: (i, 0)),
                  pl.BlockSpec((C_IMG, 384), lambda i: (0, 0))],
        out_specs=pl.BlockSpec((RB, 384), lambda i: (i, 0)),
        out_shape=jax.ShapeDtypeStruct((10240, 384), jnp.bfloat16),
    )(img_t, w)


def _prep_body(xs_ref, ys_ref, idx_ref, w_ref):
    gx = xs_ref[...]
    gy = ys_ref[...]
    x = (gx + 1.0) * ((W_IMG - 1) / 2.0)
    y = (1.0 - gy) * ((H_IMG - 1) / 2.0)
    x0 = jnp.floor(x)
    y0 = jnp.floor(y)
    wx1 = x - x0
    wx0 = 1.0 - wx1
    wy1 = y - y0
    wy0 = 1.0 - wy1
    vid = lax.broadcasted_iota(jnp.int32, (784, 128), 0) * 128 + \
        lax.broadcasted_iota(jnp.int32, (784, 128), 1)
    n = jnp.clip(vid // V, 0, N - 1)
    for t, (ix, iy, wt) in enumerate((
            (x0, y0, wx0 * wy0), (x0 + 1.0, y0, wx1 * wy0),
            (x0, y0 + 1.0, wx0 * wy1), (x0 + 1.0, y0 + 1.0, wx1 * wy1))):
        valid = ((ix >= 0.0) & (ix <= W_IMG - 1.0)
                 & (iy >= 0.0) & (iy <= H_IMG - 1.0))
        ixc = jnp.clip(ix, 0.0, W_IMG - 1.0).astype(jnp.int32)
        iyc = jnp.clip(iy, 0.0, H_IMG - 1.0).astype(jnp.int32)
        idx_ref[t] = n * (H_IMG * W_IMG) + iyc * W_IMG + ixc
        w_ref[t] = jnp.where(valid, wt, 0.0)


def _tc_prep(xs, ys):
    return pl.pallas_call(
        _prep_body,
        in_specs=[pl.BlockSpec((784, 128), lambda: (0, 0)),
                  pl.BlockSpec((784, 128), lambda: (0, 0))],
        out_specs=[pl.BlockSpec((4, 784, 128), lambda: (0, 0, 0)),
                   pl.BlockSpec((4, 784, 128), lambda: (0, 0, 0))],
        out_shape=[jax.ShapeDtypeStruct((4, 784, 128), jnp.int32),
                   jax.ShapeDtypeStruct((4, 784, 128), jnp.float32)],
    )(xs, ys)


def _va_body(t0, t1, t2, t3, w_ref, b_ref, o_ref):
    w = w_ref[...]
    acc = t0[...].astype(jnp.float32) * w[:, 0:1]
    acc += t1[...].astype(jnp.float32) * w[:, 1:2]
    acc += t2[...].astype(jnp.float32) * w[:, 2:3]
    acc += t3[...].astype(jnp.float32) * w[:, 3:4]
    o_ref[...] = jnp.maximum(acc + b_ref[...], 0.0).astype(jnp.bfloat16)


def _tc_va(taps_flat, w8, bb):
    specs = [pl.BlockSpec((RB, 128), functools.partial(
        lambda i, t: (t * NROW + i, 0), t=t)) for t in range(4)]
    return pl.pallas_call(
        _va_body,
        grid=(NROW,),
        in_specs=specs + [pl.BlockSpec((RB, 8), lambda i: (i, 0)),
                          pl.BlockSpec((1, 128), lambda i: (0, 0))],
        out_specs=pl.BlockSpec((RB, 128), lambda i: (i, 0)),
        out_shape=jax.ShapeDtypeStruct((VP, 128), jnp.bfloat16),
    )(taps_flat, taps_flat, taps_flat, taps_flat, w8, bb)


def _gconv_first_body(a_ref, p_ref, f_ref, w_ref, b_ref, o0_ref, o1_ref):
    xin = jnp.concatenate([a_ref[...], p_ref[...].astype(jnp.bfloat16),
                           f_ref[...]], axis=1)
    out = jnp.dot(xin, w_ref[...], preferred_element_type=jnp.float32)
    out += b_ref[...]
    o0_ref[...] = out[:, :HIDDEN].astype(jnp.bfloat16)
    o1_ref[...] = out[:, HIDDEN:].astype(jnp.bfloat16)


def _tc_gconv_first(va, pos8, vfeat, w, b):
    return pl.pallas_call(
        _gconv_first_body,
        grid=(NROW,),
        in_specs=[pl.BlockSpec((RB, 128), lambda i: (i, 0)),
                  pl.BlockSpec((RB, 8), lambda i: (i, 0)),
                  pl.BlockSpec((RB, 128), lambda i: (i, 0)),
                  pl.BlockSpec((264, 256), lambda i: (0, 0)),
                  pl.BlockSpec((1, 256), lambda i: (0, 0))],
        out_specs=[pl.BlockSpec((RB, 128), lambda i: (i, 0)),
                   pl.BlockSpec((RB, 128), lambda i: (i, 0))],
        out_shape=[jax.ShapeDtypeStruct((VP, 128), jnp.bfloat16),
                   jax.ShapeDtypeStruct((VP, 128), jnp.bfloat16)],
    )(va, pos8, vfeat, w, b)


def _gconv_first_s0_body(a_ref, p_ref, w_ref, b_ref, o0_ref, o1_ref):
    xin = jnp.concatenate([a_ref[...], p_ref[...].astype(jnp.bfloat16)],
                          axis=1)
    out = jnp.dot(xin, w_ref[...], preferred_element_type=jnp.float32)
    out += b_ref[...]
    o0_ref[...] = out[:, :HIDDEN].astype(jnp.bfloat16)
    o1_ref[...] = out[:, HIDDEN:].astype(jnp.bfloat16)


def _tc_gconv_first_s0(va, pos8, w, b):
    return pl.pallas_call(
        _gconv_first_s0_body,
        grid=(NROW,),
        in_specs=[pl.BlockSpec((RB, 128), lambda i: (i, 0)),
                  pl.BlockSpec((RB, 8), lambda i: (i, 0)),
                  pl.BlockSpec((136, 256), lambda i: (0, 0)),
                  pl.BlockSpec((1, 256), lambda i: (0, 0))],
        out_specs=[pl.BlockSpec((RB, 128), lambda i: (i, 0)),
                   pl.BlockSpec((RB, 128), lambda i: (i, 0))],
        out_shape=[jax.ShapeDtypeStruct((VP, 128), jnp.bfloat16),
                   jax.ShapeDtypeStruct((VP, 128), jnp.bfloat16)],
    )(va, pos8, w, b)


def _va_block(t0, t1, t2, t3, w_ref, b_ref):
    w = w_ref[...]
    acc = t0[...].astype(jnp.float32) * w[:, 0:1]
    acc += t1[...].astype(jnp.float32) * w[:, 1:2]
    acc += t2[...].astype(jnp.float32) * w[:, 2:3]
    acc += t3[...].astype(jnp.float32) * w[:, 3:4]
    return jnp.maximum(acc + b_ref[...], 0.0).astype(jnp.bfloat16)


def _gconv0_s0_body(t0, t1, t2, t3, tw_ref, bb_ref, p_ref, w_ref, b_ref,
                    o0_ref, o1_ref):
    va = _va_block(t0, t1, t2, t3, tw_ref, bb_ref)
    xin = jnp.concatenate([va, p_ref[...].astype(jnp.bfloat16)], axis=1)
    out = jnp.dot(xin, w_ref[...], preferred_element_type=jnp.float32)
    out += b_ref[...]
    o0_ref[...] = out[:, :HIDDEN].astype(jnp.bfloat16)
    o1_ref[...] = out[:, HIDDEN:].astype(jnp.bfloat16)


def _gconv0_body(t0, t1, t2, t3, tw_ref, bb_ref, p_ref, f_ref, w_ref, b_ref,
                 o0_ref, o1_ref):
    va = _va_block(t0, t1, t2, t3, tw_ref, bb_ref)
    xin = jnp.concatenate([va, p_ref[...].astype(jnp.bfloat16), f_ref[...]],
                          axis=1)
    out = jnp.dot(xin, w_ref[...], preferred_element_type=jnp.float32)
    out += b_ref[...]
    o0_ref[...] = out[:, :HIDDEN].astype(jnp.bfloat16)
    o1_ref[...] = out[:, HIDDEN:].astype(jnp.bfloat16)


def _tc_gconv0(taps_flat, w8, bb, pos8, vfeat, w, b):
    tspecs = [pl.BlockSpec((RB, 128), functools.partial(
        lambda i, t: (t * NROW + i, 0), t=t)) for t in range(4)]
    common = [pl.BlockSpec((RB, 8), lambda i: (i, 0)),
              pl.BlockSpec((1, 128), lambda i: (0, 0)),
              pl.BlockSpec((RB, 8), lambda i: (i, 0))]
    if vfeat is None:
        body = _gconv0_s0_body
        ins = common + [pl.BlockSpec((136, 256), lambda i: (0, 0)),
                        pl.BlockSpec((1, 256), lambda i: (0, 0))]
        args = (taps_flat,) * 4 + (w8, bb, pos8, w, b)
    else:
        body = _gconv0_body
        ins = common[:2] + [pl.BlockSpec((RB, 8), lambda i: (i, 0)),
                            pl.BlockSpec((RB, 128), lambda i: (i, 0)),
                            pl.BlockSpec((264, 256), lambda i: (0, 0)),
                            pl.BlockSpec((1, 256), lambda i: (0, 0))]
        args = (taps_flat,) * 4 + (w8, bb, pos8, vfeat, w, b)
    return pl.pallas_call(
        body,
        grid=(NROW,),
        in_specs=tspecs + ins,
        out_specs=[pl.BlockSpec((RB, 128), lambda i: (i, 0)),
                   pl.BlockSpec((RB, 128), lambda i: (i, 0))],
        out_shape=[jax.ShapeDtypeStruct((VP, 128), jnp.bfloat16),
                   jax.ShapeDtypeStruct((VP, 128), jnp.bfloat16)],
    )(*args)


def _gconv_mid_body(x0_ref, g_ref, p_ref, w_ref, b_ref, o0_ref, o1_ref):
    nopos = jnp.maximum(x0_ref[...].astype(jnp.float32)
                        + g_ref[...].astype(jnp.float32), 0.0)
    xin = jnp.concatenate([nopos.astype(jnp.bfloat16),
                           p_ref[...].astype(jnp.bfloat16)], axis=1)
    out = jnp.dot(xin, w_ref[...], preferred_element_type=jnp.float32)
    out += b_ref[...]
    o0_ref[...] = out[:, :HIDDEN].astype(jnp.bfloat16)
    o1_ref[...] = out[:, HIDDEN:].astype(jnp.bfloat16)


def _tc_gconv_mid(xw0, agg, pos8, w, b):
    return pl.pallas_call(
        _gconv_mid_body,
        grid=(NROW,),
        in_specs=[pl.BlockSpec((RB, 128), lambda i: (i, 0)),
                  pl.BlockSpec((RB, 128), lambda i: (i, 0)),
                  pl.BlockSpec((RB, 8), lambda i: (i, 0)),
                  pl.BlockSpec((136, 256), lambda i: (0, 0)),
                  pl.BlockSpec((1, 256), lambda i: (0, 0))],
        out_specs=[pl.BlockSpec((RB, 128), lambda i: (i, 0)),
                   pl.BlockSpec((RB, 128), lambda i: (i, 0))],
        out_shape=[jax.ShapeDtypeStruct((VP, 128), jnp.bfloat16),
                   jax.ShapeDtypeStruct((VP, 128), jnp.bfloat16)],
    )(xw0, agg, pos8, w, b)


def _offset_body(x0_ref, g_ref, p_ref, w_ref, b_ref, v_ref, np_ref):
    nopos = jnp.maximum(x0_ref[...].astype(jnp.float32)
                        + g_ref[...].astype(jnp.float32), 0.0)
    np_ref[...] = nopos.astype(jnp.bfloat16)
    xin = jnp.concatenate([nopos.astype(jnp.bfloat16),
                           p_ref[...].astype(jnp.bfloat16)], axis=1)
    out = jnp.dot(xin, w_ref[...], preferred_element_type=jnp.float32)
    v_ref[...] = p_ref[...] + jnp.tanh(out + b_ref[...])


def _tc_offset(xw0, agg, pos8, w, b):
    return pl.pallas_call(
        _offset_body,
        grid=(NROW,),
        in_specs=[pl.BlockSpec((RB, 128), lambda i: (i, 0)),
                  pl.BlockSpec((RB, 128), lambda i: (i, 0)),
                  pl.BlockSpec((RB, 8), lambda i: (i, 0)),
                  pl.BlockSpec((136, 8), lambda i: (0, 0)),
                  pl.BlockSpec((1, 8), lambda i: (0, 0))],
        out_specs=[pl.BlockSpec((RB, 8), lambda i: (i, 0)),
                   pl.BlockSpec((RB, 128), lambda i: (i, 0))],
        out_shape=[jax.ShapeDtypeStruct((VP, 8), jnp.float32),
                   jax.ShapeDtypeStruct((VP, 128), jnp.bfloat16)],
    )(xw0, agg, pos8, w, b)


def _sc_mesh():
    return plsc.VectorSubcoreMesh(core_axis_name="c", subcore_axis_name="s",
                                  num_cores=2, num_subcores=16)


def _sc_taps_body(table_hbm, idx_hbm, out_hbm, idx_v, rows_v, table_spm,
                  sem_g, sem_w):
    cid = lax.axis_index("c")
    sid = lax.axis_index("s")
    wid = sid * 2 + cid
    pltpu.sync_copy(table_hbm.at[pl.ds(sid * 640, 640)],
                    table_spm.at[pl.ds(sid * 640, 640)])
    plsc.subcore_barrier()
    base = wid * TAP_ROWS_PER_TILE
    pltpu.sync_copy(idx_hbm.at[pl.ds(base, TAP_ROWS_PER_TILE)], idx_v)
    MBR = 224
    NB = TAP_ROWS_PER_TILE // MBR
    NS = 3

    def fire_gather(j):
        pltpu.async_copy(table_spm.at[idx_v.at[pl.ds(j * MBR, MBR)]],
                         rows_v.at[lax.rem(j, NS)], sem_g)

    def body(j, _):
        @pl.when((j >= 2) & (j - 2 < NB))
        def _():
            pltpu.make_async_copy(
                rows_v.at[lax.rem(j, NS)],
                out_hbm.at[pl.ds(base + (j - 2) * MBR, MBR)], sem_w).wait()

        @pl.when(j + 2 < NB)
        def _():
            fire_gather(j + 2)

        @pl.when(j < NB)
        def _():
            slot = lax.rem(j, NS)
            pltpu.make_async_copy(table_spm.at[idx_v.at[pl.ds(j * MBR, MBR)]],
                                  rows_v.at[slot], sem_g).wait()
            pltpu.async_copy(rows_v.at[slot],
                             out_hbm.at[pl.ds(base + j * MBR, MBR)], sem_w)
        return 0

    for p in range(2):
        fire_gather(p)
    lax.fori_loop(0, NB + 2, body, 0, unroll=False)


def _sc_gather_taps(img_proj, tap_idx):
    kfn = pl.kernel(
        _sc_taps_body,
        out_type=jax.ShapeDtypeStruct((TAPS_TOT, 128), jnp.bfloat16),
        mesh=_sc_mesh(),
        scratch_types=[
            pltpu.VMEM((TAP_ROWS_PER_TILE,), jnp.int32),
            pltpu.VMEM((3, 224, 128), jnp.bfloat16),
            pltpu.VMEM_SHARED((10240, 128), jnp.bfloat16),
            pltpu.SemaphoreType.DMA,
            pltpu.SemaphoreType.DMA,
        ],
        compiler_params=pltpu.CompilerParams(use_tc_tiling_on_sc=False),
    )
    return kfn(img_proj, tap_idx)


def _sc_agg_body(vw1r_hbm, gidx_hbm, sidx_hbm, out_hbm,
                 gbuf, ibuf, sbuf, rows_v, zbuf, acc,
                 sem_i, sem_g, sem_s, sem_z):
    cid = lax.axis_index("c")
    sid = lax.axis_index("s")
    ebase = sid * E_PER_TILE
    sbase = sid * (E_PER_TILE // EB)
    KPB = EBATCH // EB

    def zb(i, _):
        zbuf[i, :] = jnp.zeros((32,), jnp.bfloat16)
        return 0
    lax.fori_loop(0, ZCHUNK, zb, 0, unroll=False)

    def fire_idx(j):
        pltpu.async_copy(gidx_hbm.at[pl.ds(ebase + j * EBATCH, EBATCH)],
                         gbuf.at[lax.rem(j, 3)], sem_i)
        pltpu.async_copy(sidx_hbm.at[pl.ds(sbase + j * KPB, KPB)],
                         sbuf.at[lax.rem(j, 5)], sem_i)

    def drain_idx(j):
        pltpu.make_async_copy(gidx_hbm.at[pl.ds(0, EBATCH)],
                              gbuf.at[lax.rem(j, 3)], sem_i).wait()
        pltpu.make_async_copy(sidx_hbm.at[pl.ds(0, KPB)],
                              sbuf.at[lax.rem(j, 5)], sem_i).wait()

    def fire_gathers(j):
        for k in range(KPB):
            pltpu.async_copy(
                vw1r_hbm.at[ibuf.at[lax.rem(j, 3), k]],
                rows_v.at[lax.rem(j, 4), pl.ds(k * EB, EB)], sem_g)

    def drain_gathers(j):
        for k in range(KPB):
            pltpu.make_async_copy(
                vw1r_hbm.at[ibuf.at[lax.rem(j, 3), k]],
                rows_v.at[lax.rem(j, 4), pl.ds(k * EB, EB)], sem_g).wait()

    def fire_scatters(j):
        for k in range(KPB):
            pltpu.async_copy(
                rows_v.at[lax.rem(j, 4), pl.ds(k * EB, EB)],
                acc.at[sbuf.at[lax.rem(j, 5), k]], sem_s, add=True)

    def drain_scatters(j):
        for k in range(KPB):
            pltpu.make_async_copy(
                rows_v.at[lax.rem(j, 4), pl.ds(k * EB, EB)],
                acc.at[sbuf.at[lax.rem(j, 5), k]], sem_s).wait()

    def do_chunk(f):
        for z in range(VPT // ZCHUNK):
            pltpu.async_copy(
                zbuf, acc.at[pl.ds(sid * VPT + z * ZCHUNK, ZCHUNK)], sem_z)
        for z in range(VPT // ZCHUNK):
            pltpu.make_async_copy(
                zbuf, acc.at[pl.ds(sid * VPT + z * ZCHUNK, ZCHUNK)],
                sem_z).wait()
        plsc.subcore_barrier()

        def batch(j, _):
            @pl.when((j >= 3) & (j - 3 < NBATCH))
            def _():
                drain_scatters(j - 3)

            @pl.when(j + 2 < NBATCH)
            def _():
                fire_idx(j + 2)

            @pl.when(j < NBATCH)
            def _():
                drain_idx(j)
                for k in range(KPB):
                    for i in range(EB // 16):
                        g = gbuf[lax.rem(j, 3), pl.ds(k * EB + i * 16, 16)]
                        ibuf[lax.rem(j, 3), k, pl.ds(i * 16, 16)] = \
                            g * 4 + f
                fire_gathers(j)

            @pl.when((j >= 2) & (j - 2 < NBATCH))
            def _():
                drain_gathers(j - 2)
                fire_scatters(j - 2)
            return 0

        fire_idx(0)
        fire_idx(1)
        lax.fori_loop(0, NBATCH + 3, batch, 0, unroll=False)
        plsc.subcore_barrier()
        pltpu.sync_copy(
            acc.at[pl.ds(sid * VPT, VPT)],
            out_hbm.at[pl.ds(sid * VPT, VPT), pl.ds(f * 32, 32)])
        plsc.subcore_barrier()

    for fi in range(2):
        do_chunk(cid * 2 + fi)


def _sc_edge_agg(vw1, gidx, sidx2d):
    kfn = pl.kernel(
        _sc_agg_body,
        out_type=jax.ShapeDtypeStruct((VP, 128), jnp.bfloat16),
        mesh=_sc_mesh(),
        scratch_types=[
            pltpu.VMEM((3, EBATCH), jnp.int32),
            pltpu.VMEM((3, EBATCH // EB, EB), jnp.int32),
            pltpu.VMEM((5, EBATCH // EB, EB), jnp.int32),
            pltpu.VMEM((4, EBATCH, 32), jnp.bfloat16),
            pltpu.VMEM((ZCHUNK, 32), jnp.bfloat16),
            pltpu.VMEM_SHARED((ACC_ROWS, 32), jnp.bfloat16),
            pltpu.SemaphoreType.DMA,
            pltpu.SemaphoreType.DMA,
            pltpu.SemaphoreType.DMA,
            pltpu.SemaphoreType.DMA,
        ],
        compiler_params=pltpu.CompilerParams(use_tc_tiling_on_sc=False),
    )
    return kfn(vw1.reshape(VP * 4, 32), gidx, sidx2d)


def _pack_gconv_w(p, first_with_feats):
    w0W, w0b = p['w0']
    w1W, w1b = p['w1']
    w0t, w1t = w0W.T, w1W.T
    if first_with_feats:
        def arrange(wt):
            return jnp.concatenate([
                wt[:HIDDEN], wt[HIDDEN:HIDDEN + 3],
                jnp.zeros((5, HIDDEN), jnp.float32),
                wt[HIDDEN + 3:]], axis=0)
    else:
        def arrange(wt):
            return jnp.concatenate([
                wt[:HIDDEN], wt[HIDDEN:HIDDEN + 3],
                jnp.zeros((5, HIDDEN), jnp.float32)], axis=0)
    w = jnp.concatenate([arrange(w0t), arrange(w1t)], axis=1)
    b = jnp.concatenate([w0b, w1b]).reshape(1, 256)
    return w.astype(jnp.bfloat16), b


def _pack_offset_w(p):
    oW, ob = p['vert_offset']
    ot = oW.T
    w = jnp.concatenate([ot[:HIDDEN], ot[HIDDEN:HIDDEN + 3],
                         jnp.zeros((5, 3), jnp.float32)], axis=0)
    w = jnp.concatenate([w, jnp.zeros((136, 5), jnp.float32)], axis=1)
    b = jnp.concatenate([ob, jnp.zeros((5,), jnp.float32)]).reshape(1, 8)
    return w.astype(jnp.bfloat16), b


def kernel(img_feats, verts_padded, edges_packed, params):
    f32 = jnp.float32
    img_t = jnp.transpose(img_feats, (0, 2, 3, 1)).reshape(N * H_IMG * W_IMG,
                                                           C_IMG)
    bws = [params['stages'][s]['bottleneck'][0].T for s in range(NUM_STAGES)]
    img_proj_all = _tc_imgproj(
        img_t, jnp.concatenate(bws, axis=1).astype(jnp.bfloat16))
    img_projs = [img_proj_all[:, s * 128:(s + 1) * 128] for s in
                 range(NUM_STAGES)]

    src = edges_packed[:, 0]
    dst = edges_packed[:, 1]
    gidx = jnp.concatenate([dst, src])
    sidx = jnp.concatenate([src, dst])
    gidx = jnp.concatenate([gidx, jnp.zeros((E2P - E2,), jnp.int32)])
    sidx = jnp.concatenate(
        [sidx, jnp.full((E2P - E2,), DUMMY_ROW, jnp.int32)])
    sidx2d = sidx.reshape(E2P // EB, EB)

    verts_flat = jnp.pad(verts_padded.reshape(VTOT, 3),
                         ((0, VP - VTOT), (0, 0)))
    pos8 = jnp.pad(verts_flat, ((0, 0), (0, 5)))

    outs = []
    vfeat = None
    for s in range(NUM_STAGES):
        sp = params['stages'][s]
        xs = pos8[:, 0].reshape(784, 128)
        ys = pos8[:, 1].reshape(784, 128)
        tap_idx, tap_w = _tc_prep(xs, ys)
        taps_flat = _sc_gather_taps(img_projs[s], tap_idx.reshape(4 * VP))
        w8 = jnp.pad(jnp.transpose(tap_w.reshape(4, VP)), ((0, 0), (0, 4)))
        bb = sp['bottleneck'][1].reshape(1, 128)

        w, b = _pack_gconv_w(sp['gconvs'][0], s > 0)
        xw0, vw1 = _tc_gconv0(taps_flat, w8, bb, pos8,
                              vfeat if s > 0 else None, w, b)
        agg = _sc_edge_agg(vw1, gidx, sidx2d)
        for gi in range(1, STAGE_DEPTH):
            w, b = _pack_gconv_w(sp['gconvs'][gi], False)
            xw0, vw1 = _tc_gconv_mid(xw0, agg, pos8, w, b)
            agg = _sc_edge_agg(vw1, gidx, sidx2d)

        ow, ob = _pack_offset_w(sp)
        verts8, vfeat = _tc_offset(xw0, agg, pos8, ow, ob)
        pos8 = verts8
        outs.append(verts8[:VTOT, :3].reshape(N, V, 3))

    return jnp.stack(outs, axis=0).astype(f32)
